# Initial kernel scaffold; baseline (speedup 1.0000x reference)
import jax
import jax.numpy as jnp
from jax.experimental import pallas as pl


def kernel(edge_index, v_feat, t_feat, v_pref, t_pref, v_W, v_b, t_W, t_b, id_emb, conf):
    raise NotImplementedError("write your pallas kernel here")


# speedup vs baseline: 13.2103x; 13.2103x over previous
import functools

import jax
import jax.numpy as jnp
from jax import lax
from jax.experimental import pallas as pl
from jax.experimental.pallas import tpu as pltpu
from jax.experimental.pallas import tpu_sc as plsc

NUM_USER = 10000
NUM_ITEM = 40000
N = NUM_USER + NUM_ITEM
E = 800000
NUM_ROUTING = 3

NW = 32
C = 256
WX = 64
NPT_F = 1600
NPAD = NPT_F * NW
NPT_R = 320
UPAD = NPT_R * NW
L1 = (E // C + 1) * C
L2 = (2 * E // C + 1) * C
NEG = -1e30

_MESH = dict(core_axis_name="c", subcore_axis_name="s", num_cores=2,
             num_subcores=16)
_CP = pltpu.CompilerParams(needs_layout_passes=False, use_tc_tiling_on_sc=False)

_f32 = jnp.float32
_i32 = jnp.int32


def _wid():
    return lax.axis_index("s") * 2 + lax.axis_index("c")


def _extract(ref, j):
    return ref[pl.ds(j, 16)][0]


def _bcast(x):
    return jnp.full((16,), x, _f32)


def _leaky_v(x):
    return jnp.where(x >= 0.0, x, 0.01 * x)


def _gat_routing(x_cat, srcs, dsts, offs):
    @functools.partial(
        pl.kernel,
        out_type=jax.ShapeDtypeStruct((UPAD, 128), _f32),
        mesh=plsc.VectorSubcoreMesh(**_MESH),
        compiler_params=_CP,
        scratch_types=[
            pltpu.VMEM((64,), _i32),
            pltpu.VMEM((C,), _i32),
            pltpu.VMEM((C + 16,), _i32),
            pltpu.VMEM((C, 128), _f32),
            pltpu.VMEM((WX, 128), _f32),
            pltpu.VMEM((WX, 128), _f32),
            pltpu.SemaphoreType.DMA,
        ],
    )
    def k(x_hbm, src_hbm, dst_hbm, off_hbm, out_hbm,
          off_v, src_v, dst_v, rows_v, xw_v, ow_v, sem):
        w = _wid()
        n0 = w * NPT_R
        n1 = n0 + NPT_R
        pltpu.sync_copy(off_hbm.at[pl.ds(0, 64)], off_v)
        e0 = _extract(off_v, w)
        e1 = _extract(off_v, w + 1)
        c_lo = e0 // C
        c_hi = (e1 + C - 1) // C

        zero16 = jnp.zeros((16,), _f32)

        def open_x(w0):
            wa = pl.multiple_of(w0, 8)
            pltpu.sync_copy(x_hbm.at[pl.ds(wa, WX), pl.ds(0, 128)], xw_v)

        def zero_out():
            def zb(r, t):
                for kk in range(8):
                    ow_v[r, pl.ds(kk * 16, 16)] = zero16
                return t
            lax.fori_loop(0, WX, zb, 0)

        def close_out(w0):
            @pl.when(w0 < n1)
            def _():
                wa = pl.multiple_of(w0, 8)
                pltpu.sync_copy(ow_v, out_hbm.at[pl.ds(wa, WX), pl.ds(0, 128)])
            zero_out()

        def flush(d_cur, wx0, s_v, s_t, a):
            r = d_cur - wx0
            inv_v = _bcast(1.0) / (s_v + 1e-16)
            inv_t = _bcast(1.0) / (s_t + 1e-16)
            for kk in range(4):
                ow_v[r, pl.ds(kk * 16, 16)] = a[kk] * inv_v
            for kk in range(4, 8):
                ow_v[r, pl.ds(kk * 16, 16)] = a[kk] * inv_t

        open_x(n0)
        zero_out()

        def chunk_body(c, carry):
            cc = pl.multiple_of(c * C, 8)
            pltpu.sync_copy(src_hbm.at[pl.ds(cc, C)], src_v)
            pltpu.sync_copy(dst_hbm.at[pl.ds(cc, C)], dst_v.at[pl.ds(0, C)])
            pltpu.async_copy(x_hbm.at[src_v], rows_v, sem).wait()
            j_lo = jnp.maximum(e0 - c * C, 0)
            j_hi = jnp.minimum(e1 - c * C, C)

            def edge_body(j, ecarry):
                (d_cur, wx0, m_v, s_v, m_t, s_t,
                 a0, a1, a2, a3, a4, a5, a6, a7) = ecarry
                d = _extract(dst_v, j)
                is_new = d != d_cur

                def do_flush(tok):
                    flush(d_cur, wx0, s_v, s_t,
                          (a0, a1, a2, a3, a4, a5, a6, a7))
                    return tok

                lax.cond(is_new & (d_cur >= n0), do_flush, lambda t: t, 0)

                def adv(_, w0):
                    close_out(w0)
                    w0 = w0 + WX
                    open_x(w0)
                    return w0
                n_adv = jnp.maximum(d - wx0, 0) // WX
                wx0 = lax.fori_loop(0, n_adv, adv, wx0)

                sel = jnp.full((16,), is_new)
                m_v = jnp.where(sel, _bcast(NEG), m_v)
                m_t = jnp.where(sel, _bcast(NEG), m_t)
                s_v = jnp.where(sel, zero16, s_v)
                s_t = jnp.where(sel, zero16, s_t)
                a0 = jnp.where(sel, zero16, a0)
                a1 = jnp.where(sel, zero16, a1)
                a2 = jnp.where(sel, zero16, a2)
                a3 = jnp.where(sel, zero16, a3)
                a4 = jnp.where(sel, zero16, a4)
                a5 = jnp.where(sel, zero16, a5)
                a6 = jnp.where(sel, zero16, a6)
                a7 = jnp.where(sel, zero16, a7)
                d_cur = d

                r = jnp.maximum(d - wx0, 0)
                r0 = rows_v[j, pl.ds(0, 16)]
                r1 = rows_v[j, pl.ds(16, 16)]
                r2 = rows_v[j, pl.ds(32, 16)]
                r3 = rows_v[j, pl.ds(48, 16)]
                r4 = rows_v[j, pl.ds(64, 16)]
                r5 = rows_v[j, pl.ds(80, 16)]
                r6 = rows_v[j, pl.ds(96, 16)]
                r7 = rows_v[j, pl.ds(112, 16)]
                dv = (r0 * xw_v[r, pl.ds(0, 16)] + r1 * xw_v[r, pl.ds(16, 16)]
                      + r2 * xw_v[r, pl.ds(32, 16)]
                      + r3 * xw_v[r, pl.ds(48, 16)])
                dt = (r4 * xw_v[r, pl.ds(64, 16)] + r5 * xw_v[r, pl.ds(80, 16)]
                      + r6 * xw_v[r, pl.ds(96, 16)]
                      + r7 * xw_v[r, pl.ds(112, 16)])
                valid = (j >= j_lo) & (j < j_hi)
                av = jnp.where(valid, jnp.sum(dv), NEG)
                at = jnp.where(valid, jnp.sum(dt), NEG)
                av_b = _bcast(av)
                at_b = _bcast(at)

                mnv = jnp.maximum(m_v, av_b)
                mnt = jnp.maximum(m_t, at_b)
                scale_v = jnp.exp(m_v - mnv)
                scale_t = jnp.exp(m_t - mnt)
                p_v = jnp.exp(av_b - mnv)
                p_t = jnp.exp(at_b - mnt)
                s_v = s_v * scale_v + p_v
                s_t = s_t * scale_t + p_t
                a0 = a0 * scale_v + p_v * r0
                a1 = a1 * scale_v + p_v * r1
                a2 = a2 * scale_v + p_v * r2
                a3 = a3 * scale_v + p_v * r3
                a4 = a4 * scale_t + p_t * r4
                a5 = a5 * scale_t + p_t * r5
                a6 = a6 * scale_t + p_t * r6
                a7 = a7 * scale_t + p_t * r7
                return (d_cur, wx0, mnv, s_v, mnt, s_t,
                        a0, a1, a2, a3, a4, a5, a6, a7)

            return lax.fori_loop(j_lo, j_hi, edge_body, carry)

        init = (jnp.int32(-1), n0,
                _bcast(NEG), zero16, _bcast(NEG), zero16,
                zero16, zero16, zero16, zero16,
                zero16, zero16, zero16, zero16)
        carry = lax.fori_loop(c_lo, c_hi, chunk_body, init)
        (d_cur, wx0, m_v, s_v, m_t, s_t,
         a0, a1, a2, a3, a4, a5, a6, a7) = carry

        def do_flush(tok):
            flush(d_cur, wx0, s_v, s_t, (a0, a1, a2, a3, a4, a5, a6, a7))
            return tok
        lax.cond(d_cur >= n0, do_flush, lambda t: t, 0)

        def drain(_, w0):
            close_out(w0)
            return w0 + WX
        lax.fori_loop(0, (n1 - wx0) // WX, drain, wx0)

    return k(x_cat, srcs, dsts, offs)


def _gat_final(x_cat, srcs, dsts, offs):
    @functools.partial(
        pl.kernel,
        out_type=(
            jax.ShapeDtypeStruct((NPAD, 128), _f32),
            jax.ShapeDtypeStruct((NPAD, 16), _f32),
            jax.ShapeDtypeStruct((L2,), _f32),
            jax.ShapeDtypeStruct((L2,), _f32),
        ),
        mesh=plsc.VectorSubcoreMesh(**_MESH),
        compiler_params=_CP,
        scratch_types=[
            pltpu.VMEM((64,), _i32),
            pltpu.VMEM((C,), _i32),
            pltpu.VMEM((C + 16,), _i32),
            pltpu.VMEM((C, 128), _f32),
            pltpu.VMEM((WX, 128), _f32),
            pltpu.VMEM((WX, 16), _f32),
            pltpu.VMEM((C,), _f32),
            pltpu.VMEM((C,), _f32),
            pltpu.SemaphoreType.DMA,
        ],
    )
    def k(x_hbm, src_hbm, dst_hbm, off_hbm,
          rep_hbm, ms_hbm, av_hbm, at_hbm,
          off_v, src_v, dst_v, rows_v, xw_v, msw_v, av_v, at_v, sem):
        w = _wid()
        n0 = w * NPT_F
        n1 = n0 + NPT_F
        pltpu.sync_copy(off_hbm.at[pl.ds(0, 64)], off_v)
        e0 = _extract(off_v, w)
        e1 = _extract(off_v, w + 1)
        c_lo = e0 // C
        c_hi = (e1 + C - 1) // C
        aligned0 = e0 == c_lo * C

        zero16 = jnp.zeros((16,), _f32)
        iota = lax.iota(_i32, 16)

        def open_x(w0):
            wa = pl.multiple_of(w0, 8)
            pltpu.sync_copy(x_hbm.at[pl.ds(wa, WX), pl.ds(0, 128)], xw_v)

        def zero_ms():
            def zb(r, t):
                msw_v[r, pl.ds(0, 16)] = zero16
                return t
            lax.fori_loop(0, WX, zb, 0)

        def close_win(w0):
            @pl.when(w0 < n1)
            def _():
                wa = pl.multiple_of(w0, 8)
                pltpu.sync_copy(xw_v, rep_hbm.at[pl.ds(wa, WX), pl.ds(0, 128)])
                pltpu.sync_copy(msw_v, ms_hbm.at[pl.ds(wa, WX), pl.ds(0, 16)])

        def flush(d_cur, wx0, m_v, s_v, m_t, s_t, a):
            r = d_cur - wx0
            inv_v = _bcast(1.0) / (s_v + 1e-16)
            inv_t = _bcast(1.0) / (s_t + 1e-16)
            for kk in range(4):
                x_row = xw_v[r, pl.ds(kk * 16, 16)]
                xw_v[r, pl.ds(kk * 16, 16)] = x_row + _leaky_v(a[kk] * inv_v)
            for kk in range(4, 8):
                x_row = xw_v[r, pl.ds(kk * 16, 16)]
                xw_v[r, pl.ds(kk * 16, 16)] = x_row + _leaky_v(a[kk] * inv_t)
            msrow = jnp.where(iota == 0, m_v,
                              jnp.where(iota == 1, s_v,
                                        jnp.where(iota == 2, m_t, s_t)))
            msw_v[r, pl.ds(0, 16)] = msrow

        open_x(n0)
        zero_ms()

        def chunk_body(c, carry):
            cc = pl.multiple_of(c * C, 8)
            pltpu.sync_copy(src_hbm.at[pl.ds(cc, C)], src_v)
            pltpu.sync_copy(dst_hbm.at[pl.ds(cc, C)], dst_v.at[pl.ds(0, C)])
            pltpu.async_copy(x_hbm.at[src_v], rows_v, sem).wait()
            j_lo = jnp.maximum(e0 - c * C, 0)
            j_hi = jnp.minimum(e1 - c * C, C)

            def edge_body(j, ecarry):
                (d_cur, wx0, m_v, s_v, m_t, s_t,
                 a0, a1, a2, a3, a4, a5, a6, a7, av_acc, at_acc) = ecarry
                d = _extract(dst_v, j)
                is_new = d != d_cur

                def do_flush(tok):
                    flush(d_cur, wx0, m_v, s_v, m_t, s_t,
                          (a0, a1, a2, a3, a4, a5, a6, a7))
                    return tok

                lax.cond(is_new & (d_cur >= n0) & (d_cur < n1),
                         do_flush, lambda t: t, 0)

                def adv(_, w0):
                    close_win(w0)
                    w0 = w0 + WX
                    open_x(w0)
                    zero_ms()
                    return w0
                n_adv = jnp.maximum(d - wx0, 0) // WX
                wx0 = lax.fori_loop(0, n_adv, adv, wx0)

                sel = jnp.full((16,), is_new)
                m_v = jnp.where(sel, _bcast(NEG), m_v)
                m_t = jnp.where(sel, _bcast(NEG), m_t)
                s_v = jnp.where(sel, zero16, s_v)
                s_t = jnp.where(sel, zero16, s_t)
                a0 = jnp.where(sel, zero16, a0)
                a1 = jnp.where(sel, zero16, a1)
                a2 = jnp.where(sel, zero16, a2)
                a3 = jnp.where(sel, zero16, a3)
                a4 = jnp.where(sel, zero16, a4)
                a5 = jnp.where(sel, zero16, a5)
                a6 = jnp.where(sel, zero16, a6)
                a7 = jnp.where(sel, zero16, a7)
                d_cur = d

                r = jnp.maximum(d - wx0, 0)
                r0 = rows_v[j, pl.ds(0, 16)]
                r1 = rows_v[j, pl.ds(16, 16)]
                r2 = rows_v[j, pl.ds(32, 16)]
                r3 = rows_v[j, pl.ds(48, 16)]
                r4 = rows_v[j, pl.ds(64, 16)]
                r5 = rows_v[j, pl.ds(80, 16)]
                r6 = rows_v[j, pl.ds(96, 16)]
                r7 = rows_v[j, pl.ds(112, 16)]
                dv = (r0 * xw_v[r, pl.ds(0, 16)] + r1 * xw_v[r, pl.ds(16, 16)]
                      + r2 * xw_v[r, pl.ds(32, 16)]
                      + r3 * xw_v[r, pl.ds(48, 16)])
                dt = (r4 * xw_v[r, pl.ds(64, 16)] + r5 * xw_v[r, pl.ds(80, 16)]
                      + r6 * xw_v[r, pl.ds(96, 16)]
                      + r7 * xw_v[r, pl.ds(112, 16)])
                alpha_v = jnp.sum(dv)
                alpha_t = jnp.sum(dt)

                lane = j - (j // 16) * 16
                base = pl.multiple_of(j - lane, 16)
                av_acc = jnp.where(iota == lane, _bcast(alpha_v), av_acc)
                at_acc = jnp.where(iota == lane, _bcast(alpha_t), at_acc)
                av_v[pl.ds(base, 16)] = av_acc
                at_v[pl.ds(base, 16)] = at_acc

                valid = (j >= j_lo) & (j < j_hi)
                av = jnp.where(valid, alpha_v, NEG)
                at = jnp.where(valid, alpha_t, NEG)
                av_b = _bcast(av)
                at_b = _bcast(at)

                mnv = jnp.maximum(m_v, av_b)
                mnt = jnp.maximum(m_t, at_b)
                scale_v = jnp.exp(m_v - mnv)
                scale_t = jnp.exp(m_t - mnt)
                p_v = jnp.exp(av_b - mnv)
                p_t = jnp.exp(at_b - mnt)
                s_v = s_v * scale_v + p_v
                s_t = s_t * scale_t + p_t
                a0 = a0 * scale_v + p_v * r0
                a1 = a1 * scale_v + p_v * r1
                a2 = a2 * scale_v + p_v * r2
                a3 = a3 * scale_v + p_v * r3
                a4 = a4 * scale_t + p_t * r4
                a5 = a5 * scale_t + p_t * r5
                a6 = a6 * scale_t + p_t * r6
                a7 = a7 * scale_t + p_t * r7
                return (d_cur, wx0, mnv, s_v, mnt, s_t,
                        a0, a1, a2, a3, a4, a5, a6, a7, av_acc, at_acc)

            carry = lax.fori_loop(j_lo, C, edge_body, carry)

            @pl.when((c > c_lo) | aligned0)
            def _():
                pltpu.sync_copy(av_v, av_hbm.at[pl.ds(cc, C)])
                pltpu.sync_copy(at_v, at_hbm.at[pl.ds(cc, C)])
            return carry

        init = (jnp.int32(-1), n0,
                _bcast(NEG), zero16, _bcast(NEG), zero16,
                zero16, zero16, zero16, zero16,
                zero16, zero16, zero16, zero16,
                zero16, zero16)
        carry = lax.fori_loop(c_lo, c_hi, chunk_body, init)
        (d_cur, wx0, m_v, s_v, m_t, s_t,
         a0, a1, a2, a3, a4, a5, a6, a7, _av, _at) = carry

        def do_flush(tok):
            flush(d_cur, wx0, m_v, s_v, m_t, s_t,
                  (a0, a1, a2, a3, a4, a5, a6, a7))
            return tok
        lax.cond((d_cur >= n0) & (d_cur < n1), do_flush, lambda t: t, 0)

        def drain(_, w0):
            close_win(w0)
            w0 = w0 + WX
            @pl.when(w0 < n1)
            def _():
                open_x(w0)
                zero_ms()
            return w0
        lax.fori_loop(0, (n1 - wx0) // WX, drain, wx0)

    return k(x_cat, srcs, dsts, offs)


def _edge_weights(av, at, ms, conf16, srcs, dsts):
    total_chunks = L2 // C
    kpt = (total_chunks + NW - 1) // NW

    @functools.partial(
        pl.kernel,
        out_type=jax.ShapeDtypeStruct((L2,), _f32),
        mesh=plsc.VectorSubcoreMesh(**_MESH),
        compiler_params=_CP,
        scratch_types=[
            pltpu.VMEM((C,), _i32),
            pltpu.VMEM((C,), _i32),
            pltpu.VMEM((C,), _f32),
            pltpu.VMEM((C,), _f32),
            pltpu.VMEM((C, 16), _f32),
            pltpu.VMEM((C, 16), _f32),
            pltpu.VMEM((C,), _f32),
            pltpu.SemaphoreType.DMA,
            pltpu.SemaphoreType.DMA,
        ],
    )
    def k(av_hbm, at_hbm, ms_hbm, cf_hbm, src_hbm, dst_hbm, w_hbm,
          src_v, dst_v, av_v, at_v, ms_v, cf_v, w_v, sem1, sem2):
        w = _wid()
        c_beg = w * kpt
        c_end = jnp.minimum(c_beg + kpt, total_chunks)
        iota = lax.iota(_i32, 16)

        def chunk_body(c, tok):
            cc = pl.multiple_of(c * C, 8)
            pltpu.sync_copy(src_hbm.at[pl.ds(cc, C)], src_v)
            pltpu.sync_copy(dst_hbm.at[pl.ds(cc, C)], dst_v)
            pltpu.sync_copy(av_hbm.at[pl.ds(cc, C)], av_v)
            pltpu.sync_copy(at_hbm.at[pl.ds(cc, C)], at_v)
            cp1 = pltpu.async_copy(ms_hbm.at[dst_v], ms_v, sem1)
            cp2 = pltpu.async_copy(cf_hbm.at[src_v], cf_v, sem2)
            cp1.wait()
            cp2.wait()
            for g in range(C // 16):
                rows = iota + g * 16
                mv = plsc.load_gather(ms_v, [rows, jnp.zeros((16,), _i32)])
                sv = plsc.load_gather(ms_v, [rows, jnp.full((16,), 1, _i32)])
                mt = plsc.load_gather(ms_v, [rows, jnp.full((16,), 2, _i32)])
                st = plsc.load_gather(ms_v, [rows, jnp.full((16,), 3, _i32)])
                c0 = plsc.load_gather(cf_v, [rows, jnp.zeros((16,), _i32)])
                c1 = plsc.load_gather(cf_v, [rows, jnp.full((16,), 1, _i32)])
                avv = av_v[pl.ds(g * 16, 16)]
                att = at_v[pl.ds(g * 16, 16)]
                a_v = jnp.exp(avv - mv) / (sv + 1e-16)
                a_t = jnp.exp(att - mt) / (st + 1e-16)
                wv = jnp.maximum(a_v * c0, a_t * c1)
                w_v[pl.ds(g * 16, 16)] = jnp.maximum(wv, 0.0)
            pltpu.sync_copy(w_v, w_hbm.at[pl.ds(cc, C)])
            return tok

        lax.fori_loop(c_beg, c_end, chunk_body, 0)

    return k(av, at, ms, conf16, srcs, dsts)


def _sage(x, srcs, dsts, wts, offs):
    @functools.partial(
        pl.kernel,
        out_type=jax.ShapeDtypeStruct((NPAD, 64), _f32),
        mesh=plsc.VectorSubcoreMesh(**_MESH),
        compiler_params=_CP,
        scratch_types=[
            pltpu.VMEM((64,), _i32),
            pltpu.VMEM((C,), _i32),
            pltpu.VMEM((C + 16,), _i32),
            pltpu.VMEM((C + 16,), _f32),
            pltpu.VMEM((C, 64), _f32),
            pltpu.VMEM((WX, 64), _f32),
            pltpu.SemaphoreType.DMA,
        ],
    )
    def k(x_hbm, src_hbm, dst_hbm, wt_hbm, off_hbm, out_hbm,
          off_v, src_v, dst_v, wt_v, rows_v, ow_v, sem):
        w = _wid()
        n0 = w * NPT_F
        n1 = n0 + NPT_F
        pltpu.sync_copy(off_hbm.at[pl.ds(0, 64)], off_v)
        e0 = _extract(off_v, w)
        e1 = _extract(off_v, w + 1)
        c_lo = e0 // C
        c_hi = (e1 + C - 1) // C

        zero16 = jnp.zeros((16,), _f32)

        def zero_out():
            def zb(r, t):
                for kk in range(4):
                    ow_v[r, pl.ds(kk * 16, 16)] = zero16
                return t
            lax.fori_loop(0, WX, zb, 0)

        def close_out(w0):
            @pl.when(w0 < n1)
            def _():
                wa = pl.multiple_of(w0, 8)
                pltpu.sync_copy(ow_v, out_hbm.at[pl.ds(wa, WX), pl.ds(0, 64)])
            zero_out()

        def flush(d_cur, wx0, a):
            r = d_cur - wx0
            for kk in range(4):
                ow_v[r, pl.ds(kk * 16, 16)] = _leaky_v(a[kk])

        zero_out()

        def chunk_body(c, carry):
            cc = pl.multiple_of(c * C, 8)
            pltpu.sync_copy(src_hbm.at[pl.ds(cc, C)], src_v)
            pltpu.sync_copy(dst_hbm.at[pl.ds(cc, C)], dst_v.at[pl.ds(0, C)])
            pltpu.sync_copy(wt_hbm.at[pl.ds(cc, C)], wt_v.at[pl.ds(0, C)])
            pltpu.async_copy(x_hbm.at[src_v], rows_v, sem).wait()
            j_lo = jnp.maximum(e0 - c * C, 0)
            j_hi = jnp.minimum(e1 - c * C, C)

            def edge_body(j, ecarry):
                d_cur, wx0, a0, a1, a2, a3 = ecarry
                d = _extract(dst_v, j)
                is_new = d != d_cur

                def do_flush(tok):
                    flush(d_cur, wx0, (a0, a1, a2, a3))
                    return tok
                lax.cond(is_new & (d_cur >= n0), do_flush, lambda t: t, 0)

                def adv(_, w0):
                    close_out(w0)
                    return w0 + WX
                n_adv = jnp.maximum(d - wx0, 0) // WX
                wx0 = lax.fori_loop(0, n_adv, adv, wx0)

                sel = jnp.full((16,), is_new)
                a0 = jnp.where(sel, zero16, a0)
                a1 = jnp.where(sel, zero16, a1)
                a2 = jnp.where(sel, zero16, a2)
                a3 = jnp.where(sel, zero16, a3)
                d_cur = d

                we = _bcast(wt_v[pl.ds(j, 16)][0])
                a0 = a0 + we * rows_v[j, pl.ds(0, 16)]
                a1 = a1 + we * rows_v[j, pl.ds(16, 16)]
                a2 = a2 + we * rows_v[j, pl.ds(32, 16)]
                a3 = a3 + we * rows_v[j, pl.ds(48, 16)]
                return (d_cur, wx0, a0, a1, a2, a3)

            return lax.fori_loop(j_lo, j_hi, edge_body, carry)

        init = (jnp.int32(-1), n0, zero16, zero16, zero16, zero16)
        carry = lax.fori_loop(c_lo, c_hi, chunk_body, init)
        d_cur, wx0, a0, a1, a2, a3 = carry

        def do_flush(tok):
            flush(d_cur, wx0, (a0, a1, a2, a3))
            return tok
        lax.cond(d_cur >= n0, do_flush, lambda t: t, 0)

        def drain(_, w0):
            close_out(w0)
            return w0 + WX
        lax.fori_loop(0, (n1 - wx0) // WX, drain, wx0)

    return k(x, srcs, dsts, wts, offs)


def _feats_tc(v_feat, t_feat, v_W, v_b, t_W, t_b):
    vf = jnp.pad(v_feat, ((0, 40960 - NUM_ITEM), (0, 0)))
    tf = jnp.pad(t_feat, ((0, 40960 - NUM_ITEM), (0, 0)))
    vb = jnp.broadcast_to(v_b[None, :], (8, 64))
    tb = jnp.broadcast_to(t_b[None, :], (8, 64))

    def body(vf_ref, tf_ref, vw_ref, tw_ref, vb_ref, tb_ref, o_ref):
        def half(f, wgt, b):
            y = jax.lax.dot_general(f, wgt, (((1,), (1,)), ((), ())),
                                    preferred_element_type=_f32)
            y = y + b[0:1, :]
            y = jnp.where(y >= 0, y, 0.01 * y)
            nrm = jnp.sqrt(jnp.sum(y * y, axis=1, keepdims=True))
            return y / jnp.maximum(nrm, 1e-12)
        o_ref[:, 0:64] = half(vf_ref[...], vw_ref[...], vb_ref[...])
        o_ref[:, 64:128] = half(tf_ref[...], tw_ref[...], tb_ref[...])

    return pl.pallas_call(
        body,
        grid=(40960 // 256,),
        in_specs=[
            pl.BlockSpec((256, 128), lambda i: (i, 0)),
            pl.BlockSpec((256, 128), lambda i: (i, 0)),
            pl.BlockSpec((64, 128), lambda i: (0, 0)),
            pl.BlockSpec((64, 128), lambda i: (0, 0)),
            pl.BlockSpec((8, 64), lambda i: (0, 0)),
            pl.BlockSpec((8, 64), lambda i: (0, 0)),
        ],
        out_specs=pl.BlockSpec((256, 128), lambda i: (i, 0)),
        out_shape=jax.ShapeDtypeStruct((40960, 128), _f32),
    )(vf, tf, v_W, t_W, vb, tb)


def _pref_update_tc(pref_cat, xhat):
    def body(p_ref, x_ref, o_ref):
        y = p_ref[...] + x_ref[...]
        for lo in (0, 64):
            h = y[:, lo:lo + 64]
            nrm = jnp.sqrt(jnp.sum(h * h, axis=1, keepdims=True))
            o_ref[:, lo:lo + 64] = h / jnp.maximum(nrm, 1e-12)

    return pl.pallas_call(
        body,
        grid=(UPAD // 256,),
        in_specs=[
            pl.BlockSpec((256, 128), lambda i: (i, 0)),
            pl.BlockSpec((256, 128), lambda i: (i, 0)),
        ],
        out_specs=pl.BlockSpec((256, 128), lambda i: (i, 0)),
        out_shape=jax.ShapeDtypeStruct((UPAD, 128), _f32),
    )(pref_cat, xhat)


def _l2norm_tc(x):
    def body(x_ref, o_ref):
        y = x_ref[...]
        nrm = jnp.sqrt(jnp.sum(y * y, axis=1, keepdims=True))
        o_ref[...] = y / jnp.maximum(nrm, 1e-12)

    return pl.pallas_call(
        body,
        grid=(NPAD // 256,),
        in_specs=[pl.BlockSpec((256, 64), lambda i: (i, 0))],
        out_specs=pl.BlockSpec((256, 64), lambda i: (i, 0)),
        out_shape=jax.ShapeDtypeStruct((NPAD, 64), _f32),
    )(x)


def _fuse_tc(x0, x1, x2, rep):
    def body(a_ref, b_ref, c_ref, r_ref, o_ref):
        o_ref[:, 0:64] = a_ref[...] + b_ref[...] + c_ref[...]
        o_ref[:, 64:192] = r_ref[...]

    return pl.pallas_call(
        body,
        grid=(NPAD // 256,),
        in_specs=[
            pl.BlockSpec((256, 64), lambda i: (i, 0)),
            pl.BlockSpec((256, 64), lambda i: (i, 0)),
            pl.BlockSpec((256, 64), lambda i: (i, 0)),
            pl.BlockSpec((256, 128), lambda i: (i, 0)),
        ],
        out_specs=pl.BlockSpec((256, 192), lambda i: (i, 0)),
        out_shape=jax.ShapeDtypeStruct((NPAD, 192), _f32),
    )(x0, x1, x2, rep)


def kernel(edge_index, v_feat, t_feat, v_pref, t_pref, v_W, v_b, t_W, t_b,
           id_emb, conf):
    src = edge_index[0].astype(_i32)
    dst = edge_index[1].astype(_i32)

    perm1 = jnp.argsort(dst)
    srcs1 = jnp.pad(src[perm1], (0, L1 - E)).astype(_i32)
    dsts1_u = dst[perm1]
    dsts1 = jnp.pad(dsts1_u, (0, L1 - E)).astype(_i32)
    b_r = jnp.minimum(jnp.arange(33) * NPT_R, NUM_USER)
    off_r = jnp.pad(jnp.searchsorted(dsts1_u, b_r).astype(_i32), (0, 31))

    src2 = jnp.concatenate([src, dst])
    dst2 = jnp.concatenate([dst, src])
    perm2 = jnp.argsort(dst2)
    srcs2 = jnp.pad(src2[perm2], (0, L2 - 2 * E)).astype(_i32)
    dsts2_u = dst2[perm2]
    dsts2 = jnp.pad(dsts2_u, (0, L2 - 2 * E)).astype(_i32)
    b_f = jnp.minimum(jnp.arange(33) * NPT_F, N)
    off_f = jnp.pad(jnp.searchsorted(dsts2_u, b_f).astype(_i32), (0, 31))

    feats_cat = _feats_tc(v_feat, t_feat, v_W, v_b, t_W, t_b)
    pref0 = jnp.pad(jnp.concatenate([v_pref, t_pref], axis=1),
                    ((0, UPAD - NUM_USER), (0, 0)))
    pref_cat = _pref_update_tc(pref0, jnp.zeros((UPAD, 128), _f32))

    pad_tail = jnp.zeros((NPAD - N, 128), _f32)

    for _ in range(NUM_ROUTING):
        x_cat = jnp.concatenate(
            [pref_cat[:NUM_USER], feats_cat[:NUM_ITEM], pad_tail])
        xh = _gat_routing(x_cat, srcs1, dsts1, off_r)
        pref_cat = _pref_update_tc(pref_cat, xh)

    x_cat = jnp.concatenate(
        [pref_cat[:NUM_USER], feats_cat[:NUM_ITEM], pad_tail])
    rep, ms, av, at = _gat_final(x_cat, srcs2, dsts2, off_f)

    conf16 = jnp.zeros((NPAD, 16), _f32).at[:N, 0:2].set(conf)
    wts = _edge_weights(av, at, ms, conf16, srcs2, dsts2)

    x0 = _l2norm_tc(jnp.pad(id_emb, ((0, NPAD - N), (0, 0))))
    x1 = _sage(x0, srcs2, dsts2, wts, off_f)
    x2 = _sage(x1, srcs2, dsts2, wts, off_f)

    out = _fuse_tc(x0, x1, x2, rep)
    return out[:N]


# speedup vs baseline: 13.3605x; 1.0114x over previous
import functools

import jax
import jax.numpy as jnp
from jax import lax
from jax.experimental import pallas as pl
from jax.experimental.pallas import tpu as pltpu
from jax.experimental.pallas import tpu_sc as plsc

NUM_USER = 10000
NUM_ITEM = 40000
N = NUM_USER + NUM_ITEM
E = 800000
NUM_ROUTING = 3

NW = 32
C = 256
WX = 64
NPT_F = 1600
NPAD = NPT_F * NW
NPT_R = 320
UPAD = NPT_R * NW
L1 = (E // C + 1) * C
L2 = (2 * E // C + 1) * C
NEG = -1e30

_MESH = dict(core_axis_name="c", subcore_axis_name="s", num_cores=2,
             num_subcores=16)
_CP = pltpu.CompilerParams(needs_layout_passes=False, use_tc_tiling_on_sc=False)

_f32 = jnp.float32
_i32 = jnp.int32


def _wid():
    return lax.axis_index("s") * 2 + lax.axis_index("c")


def _extract(ref, j):
    return ref[pl.ds(j, 16)][0]


def _bcast(x):
    return jnp.full((16,), x, _f32)


def _leaky_v(x):
    return jnp.where(x >= 0.0, x, 0.01 * x)


def _gat_routing(x_cat, srcs, dsts, offs):
    @functools.partial(
        pl.kernel,
        out_type=jax.ShapeDtypeStruct((UPAD, 128), _f32),
        mesh=plsc.VectorSubcoreMesh(**_MESH),
        compiler_params=_CP,
        scratch_types=[
            pltpu.VMEM((64,), _i32),
            pltpu.VMEM((C,), _i32),
            pltpu.VMEM((C + 16,), _i32),
            pltpu.VMEM((C, 128), _f32),
            pltpu.VMEM((WX, 128), _f32),
            pltpu.VMEM((WX, 128), _f32),
            pltpu.SemaphoreType.DMA,
        ],
    )
    def k(x_hbm, src_hbm, dst_hbm, off_hbm, out_hbm,
          off_v, src_v, dst_v, rows_v, xw_v, ow_v, sem):
        w = _wid()
        n0 = w * NPT_R
        n1 = n0 + NPT_R
        pltpu.sync_copy(off_hbm.at[pl.ds(0, 64)], off_v)
        e0 = _extract(off_v, w)
        e1 = _extract(off_v, w + 1)
        c_lo = e0 // C
        c_hi = (e1 + C - 1) // C

        zero16 = jnp.zeros((16,), _f32)

        def open_x(w0):
            wa = pl.multiple_of(w0, 8)
            pltpu.sync_copy(x_hbm.at[pl.ds(wa, WX), pl.ds(0, 128)], xw_v)

        def zero_out():
            def zb(r, t):
                for kk in range(8):
                    ow_v[r, pl.ds(kk * 16, 16)] = zero16
                return t
            lax.fori_loop(0, WX, zb, 0)

        def close_out(w0):
            @pl.when(w0 < n1)
            def _():
                wa = pl.multiple_of(w0, 8)
                pltpu.sync_copy(ow_v, out_hbm.at[pl.ds(wa, WX), pl.ds(0, 128)])
            zero_out()

        def flush(d_cur, wx0, s_v, s_t, a):
            r = d_cur - wx0
            inv_v = _bcast(1.0) / (s_v + 1e-16)
            inv_t = _bcast(1.0) / (s_t + 1e-16)
            for kk in range(4):
                ow_v[r, pl.ds(kk * 16, 16)] = a[kk] * inv_v
            for kk in range(4, 8):
                ow_v[r, pl.ds(kk * 16, 16)] = a[kk] * inv_t

        open_x(n0)
        zero_out()

        def chunk_body(c, carry):
            cc = pl.multiple_of(c * C, 8)
            pltpu.sync_copy(src_hbm.at[pl.ds(cc, C)], src_v)
            pltpu.sync_copy(dst_hbm.at[pl.ds(cc, C)], dst_v.at[pl.ds(0, C)])
            pltpu.async_copy(x_hbm.at[src_v], rows_v, sem).wait()
            j_lo = jnp.maximum(e0 - c * C, 0)
            j_hi = jnp.minimum(e1 - c * C, C)

            def edge_body(j, ecarry):
                (d_cur, wx0, s_v, s_t,
                 a0, a1, a2, a3, a4, a5, a6, a7) = ecarry
                d = _extract(dst_v, j)
                is_new = d != d_cur

                def do_flush(tok):
                    flush(d_cur, wx0, s_v, s_t,
                          (a0, a1, a2, a3, a4, a5, a6, a7))
                    return tok

                lax.cond(is_new & (d_cur >= n0), do_flush, lambda t: t, 0)

                def adv(_, w0):
                    close_out(w0)
                    w0 = w0 + WX
                    open_x(w0)
                    return w0
                n_adv = jnp.maximum(d - wx0, 0) // WX
                wx0 = lax.fori_loop(0, n_adv, adv, wx0)

                sel = jnp.full((16,), is_new)
                s_v = jnp.where(sel, zero16, s_v)
                s_t = jnp.where(sel, zero16, s_t)
                a0 = jnp.where(sel, zero16, a0)
                a1 = jnp.where(sel, zero16, a1)
                a2 = jnp.where(sel, zero16, a2)
                a3 = jnp.where(sel, zero16, a3)
                a4 = jnp.where(sel, zero16, a4)
                a5 = jnp.where(sel, zero16, a5)
                a6 = jnp.where(sel, zero16, a6)
                a7 = jnp.where(sel, zero16, a7)
                d_cur = d

                r = jnp.maximum(d - wx0, 0)
                r0 = rows_v[j, pl.ds(0, 16)]
                r1 = rows_v[j, pl.ds(16, 16)]
                r2 = rows_v[j, pl.ds(32, 16)]
                r3 = rows_v[j, pl.ds(48, 16)]
                r4 = rows_v[j, pl.ds(64, 16)]
                r5 = rows_v[j, pl.ds(80, 16)]
                r6 = rows_v[j, pl.ds(96, 16)]
                r7 = rows_v[j, pl.ds(112, 16)]
                dv = (r0 * xw_v[r, pl.ds(0, 16)] + r1 * xw_v[r, pl.ds(16, 16)]
                      + r2 * xw_v[r, pl.ds(32, 16)]
                      + r3 * xw_v[r, pl.ds(48, 16)])
                dt = (r4 * xw_v[r, pl.ds(64, 16)] + r5 * xw_v[r, pl.ds(80, 16)]
                      + r6 * xw_v[r, pl.ds(96, 16)]
                      + r7 * xw_v[r, pl.ds(112, 16)])
                valid = (j >= j_lo) & (j < j_hi)
                av = jnp.where(valid, jnp.sum(dv), NEG)
                at = jnp.where(valid, jnp.sum(dt), NEG)
                p_v = jnp.exp(_bcast(av))
                p_t = jnp.exp(_bcast(at))
                s_v = s_v + p_v
                s_t = s_t + p_t
                a0 = a0 + p_v * r0
                a1 = a1 + p_v * r1
                a2 = a2 + p_v * r2
                a3 = a3 + p_v * r3
                a4 = a4 + p_t * r4
                a5 = a5 + p_t * r5
                a6 = a6 + p_t * r6
                a7 = a7 + p_t * r7
                return (d_cur, wx0, s_v, s_t,
                        a0, a1, a2, a3, a4, a5, a6, a7)

            return lax.fori_loop(j_lo, j_hi, edge_body, carry)

        init = (jnp.int32(-1), n0,
                zero16, zero16,
                zero16, zero16, zero16, zero16,
                zero16, zero16, zero16, zero16)
        carry = lax.fori_loop(c_lo, c_hi, chunk_body, init)
        (d_cur, wx0, s_v, s_t,
         a0, a1, a2, a3, a4, a5, a6, a7) = carry

        def do_flush(tok):
            flush(d_cur, wx0, s_v, s_t, (a0, a1, a2, a3, a4, a5, a6, a7))
            return tok
        lax.cond(d_cur >= n0, do_flush, lambda t: t, 0)

        def drain(_, w0):
            close_out(w0)
            return w0 + WX
        lax.fori_loop(0, (n1 - wx0) // WX, drain, wx0)

    return k(x_cat, srcs, dsts, offs)


def _gat_final(x_cat, srcs, dsts, offs):
    @functools.partial(
        pl.kernel,
        out_type=(
            jax.ShapeDtypeStruct((NPAD, 128), _f32),
            jax.ShapeDtypeStruct((NPAD, 16), _f32),
            jax.ShapeDtypeStruct((L2,), _f32),
            jax.ShapeDtypeStruct((L2,), _f32),
        ),
        mesh=plsc.VectorSubcoreMesh(**_MESH),
        compiler_params=_CP,
        scratch_types=[
            pltpu.VMEM((64,), _i32),
            pltpu.VMEM((C,), _i32),
            pltpu.VMEM((C + 16,), _i32),
            pltpu.VMEM((C, 128), _f32),
            pltpu.VMEM((WX, 128), _f32),
            pltpu.VMEM((WX, 16), _f32),
            pltpu.VMEM((C,), _f32),
            pltpu.VMEM((C,), _f32),
            pltpu.SemaphoreType.DMA,
        ],
    )
    def k(x_hbm, src_hbm, dst_hbm, off_hbm,
          rep_hbm, ms_hbm, av_hbm, at_hbm,
          off_v, src_v, dst_v, rows_v, xw_v, msw_v, av_v, at_v, sem):
        w = _wid()
        n0 = w * NPT_F
        n1 = n0 + NPT_F
        pltpu.sync_copy(off_hbm.at[pl.ds(0, 64)], off_v)
        e0 = _extract(off_v, w)
        e1 = _extract(off_v, w + 1)
        c_lo = e0 // C
        c_hi = (e1 + C - 1) // C
        aligned0 = e0 == c_lo * C

        zero16 = jnp.zeros((16,), _f32)
        iota = lax.iota(_i32, 16)

        def open_x(w0):
            wa = pl.multiple_of(w0, 8)
            pltpu.sync_copy(x_hbm.at[pl.ds(wa, WX), pl.ds(0, 128)], xw_v)

        def zero_ms():
            def zb(r, t):
                msw_v[r, pl.ds(0, 16)] = zero16
                return t
            lax.fori_loop(0, WX, zb, 0)

        def close_win(w0):
            @pl.when(w0 < n1)
            def _():
                wa = pl.multiple_of(w0, 8)
                pltpu.sync_copy(xw_v, rep_hbm.at[pl.ds(wa, WX), pl.ds(0, 128)])
                pltpu.sync_copy(msw_v, ms_hbm.at[pl.ds(wa, WX), pl.ds(0, 16)])

        def flush(d_cur, wx0, s_v, s_t, a):
            r = d_cur - wx0
            inv_v = _bcast(1.0) / (s_v + 1e-16)
            inv_t = _bcast(1.0) / (s_t + 1e-16)
            for kk in range(4):
                x_row = xw_v[r, pl.ds(kk * 16, 16)]
                xw_v[r, pl.ds(kk * 16, 16)] = x_row + _leaky_v(a[kk] * inv_v)
            for kk in range(4, 8):
                x_row = xw_v[r, pl.ds(kk * 16, 16)]
                xw_v[r, pl.ds(kk * 16, 16)] = x_row + _leaky_v(a[kk] * inv_t)
            msrow = jnp.where(iota == 0, s_v, jnp.where(iota == 1, s_t, zero16))
            msw_v[r, pl.ds(0, 16)] = msrow

        open_x(n0)
        zero_ms()

        def chunk_body(c, carry):
            cc = pl.multiple_of(c * C, 8)
            pltpu.sync_copy(src_hbm.at[pl.ds(cc, C)], src_v)
            pltpu.sync_copy(dst_hbm.at[pl.ds(cc, C)], dst_v.at[pl.ds(0, C)])
            pltpu.async_copy(x_hbm.at[src_v], rows_v, sem).wait()
            j_lo = jnp.maximum(e0 - c * C, 0)
            j_hi = jnp.minimum(e1 - c * C, C)

            def edge_body(j, ecarry):
                (d_cur, wx0, s_v, s_t,
                 a0, a1, a2, a3, a4, a5, a6, a7, av_acc, at_acc) = ecarry
                d = _extract(dst_v, j)
                is_new = d != d_cur

                def do_flush(tok):
                    flush(d_cur, wx0, s_v, s_t,
                          (a0, a1, a2, a3, a4, a5, a6, a7))
                    return tok

                lax.cond(is_new & (d_cur >= n0) & (d_cur < n1),
                         do_flush, lambda t: t, 0)

                def adv(_, w0):
                    close_win(w0)
                    w0 = w0 + WX
                    open_x(w0)
                    zero_ms()
                    return w0
                n_adv = jnp.maximum(d - wx0, 0) // WX
                wx0 = lax.fori_loop(0, n_adv, adv, wx0)

                sel = jnp.full((16,), is_new)
                s_v = jnp.where(sel, zero16, s_v)
                s_t = jnp.where(sel, zero16, s_t)
                a0 = jnp.where(sel, zero16, a0)
                a1 = jnp.where(sel, zero16, a1)
                a2 = jnp.where(sel, zero16, a2)
                a3 = jnp.where(sel, zero16, a3)
                a4 = jnp.where(sel, zero16, a4)
                a5 = jnp.where(sel, zero16, a5)
                a6 = jnp.where(sel, zero16, a6)
                a7 = jnp.where(sel, zero16, a7)
                d_cur = d

                r = jnp.maximum(d - wx0, 0)
                r0 = rows_v[j, pl.ds(0, 16)]
                r1 = rows_v[j, pl.ds(16, 16)]
                r2 = rows_v[j, pl.ds(32, 16)]
                r3 = rows_v[j, pl.ds(48, 16)]
                r4 = rows_v[j, pl.ds(64, 16)]
                r5 = rows_v[j, pl.ds(80, 16)]
                r6 = rows_v[j, pl.ds(96, 16)]
                r7 = rows_v[j, pl.ds(112, 16)]
                dv = (r0 * xw_v[r, pl.ds(0, 16)] + r1 * xw_v[r, pl.ds(16, 16)]
                      + r2 * xw_v[r, pl.ds(32, 16)]
                      + r3 * xw_v[r, pl.ds(48, 16)])
                dt = (r4 * xw_v[r, pl.ds(64, 16)] + r5 * xw_v[r, pl.ds(80, 16)]
                      + r6 * xw_v[r, pl.ds(96, 16)]
                      + r7 * xw_v[r, pl.ds(112, 16)])
                alpha_v = jnp.sum(dv)
                alpha_t = jnp.sum(dt)

                lane = j - (j // 16) * 16
                lane_sel = iota == lane
                av_acc = jnp.where(lane_sel, _bcast(alpha_v), av_acc)
                at_acc = jnp.where(lane_sel, _bcast(alpha_t), at_acc)

                @pl.when(lane == 15)
                def _():
                    base = pl.multiple_of(j - 15, 16)
                    av_v[pl.ds(base, 16)] = av_acc
                    at_v[pl.ds(base, 16)] = at_acc

                valid = (j >= j_lo) & (j < j_hi)
                av = jnp.where(valid, alpha_v, NEG)
                at = jnp.where(valid, alpha_t, NEG)
                p_v = jnp.exp(_bcast(av))
                p_t = jnp.exp(_bcast(at))
                s_v = s_v + p_v
                s_t = s_t + p_t
                a0 = a0 + p_v * r0
                a1 = a1 + p_v * r1
                a2 = a2 + p_v * r2
                a3 = a3 + p_v * r3
                a4 = a4 + p_t * r4
                a5 = a5 + p_t * r5
                a6 = a6 + p_t * r6
                a7 = a7 + p_t * r7
                return (d_cur, wx0, s_v, s_t,
                        a0, a1, a2, a3, a4, a5, a6, a7, av_acc, at_acc)

            carry = lax.fori_loop(j_lo, C, edge_body, carry)

            @pl.when((c > c_lo) | aligned0)
            def _():
                pltpu.sync_copy(av_v, av_hbm.at[pl.ds(cc, C)])
                pltpu.sync_copy(at_v, at_hbm.at[pl.ds(cc, C)])
            return carry

        init = (jnp.int32(-1), n0,
                zero16, zero16,
                zero16, zero16, zero16, zero16,
                zero16, zero16, zero16, zero16,
                zero16, zero16)
        carry = lax.fori_loop(c_lo, c_hi, chunk_body, init)
        (d_cur, wx0, s_v, s_t,
         a0, a1, a2, a3, a4, a5, a6, a7, _av, _at) = carry

        def do_flush(tok):
            flush(d_cur, wx0, s_v, s_t,
                  (a0, a1, a2, a3, a4, a5, a6, a7))
            return tok
        lax.cond((d_cur >= n0) & (d_cur < n1), do_flush, lambda t: t, 0)

        def drain(_, w0):
            close_win(w0)
            w0 = w0 + WX
            @pl.when(w0 < n1)
            def _():
                open_x(w0)
                zero_ms()
            return w0
        lax.fori_loop(0, (n1 - wx0) // WX, drain, wx0)

    return k(x_cat, srcs, dsts, offs)


def _edge_weights(av, at, ms, conf16, srcs, dsts):
    total_chunks = L2 // C
    kpt = (total_chunks + NW - 1) // NW

    @functools.partial(
        pl.kernel,
        out_type=jax.ShapeDtypeStruct((L2,), _f32),
        mesh=plsc.VectorSubcoreMesh(**_MESH),
        compiler_params=_CP,
        scratch_types=[
            pltpu.VMEM((C,), _i32),
            pltpu.VMEM((C,), _i32),
            pltpu.VMEM((C,), _f32),
            pltpu.VMEM((C,), _f32),
            pltpu.VMEM((C, 16), _f32),
            pltpu.VMEM((C, 16), _f32),
            pltpu.VMEM((C,), _f32),
            pltpu.SemaphoreType.DMA,
            pltpu.SemaphoreType.DMA,
        ],
    )
    def k(av_hbm, at_hbm, ms_hbm, cf_hbm, src_hbm, dst_hbm, w_hbm,
          src_v, dst_v, av_v, at_v, ms_v, cf_v, w_v, sem1, sem2):
        w = _wid()
        c_beg = w * kpt
        c_end = jnp.minimum(c_beg + kpt, total_chunks)
        iota = lax.iota(_i32, 16)

        def chunk_body(c, tok):
            cc = pl.multiple_of(c * C, 8)
            pltpu.sync_copy(src_hbm.at[pl.ds(cc, C)], src_v)
            pltpu.sync_copy(dst_hbm.at[pl.ds(cc, C)], dst_v)
            pltpu.sync_copy(av_hbm.at[pl.ds(cc, C)], av_v)
            pltpu.sync_copy(at_hbm.at[pl.ds(cc, C)], at_v)
            cp1 = pltpu.async_copy(ms_hbm.at[dst_v], ms_v, sem1)
            cp2 = pltpu.async_copy(cf_hbm.at[src_v], cf_v, sem2)
            cp1.wait()
            cp2.wait()
            for g in range(C // 16):
                rows = iota + g * 16
                sv = plsc.load_gather(ms_v, [rows, jnp.zeros((16,), _i32)])
                st = plsc.load_gather(ms_v, [rows, jnp.full((16,), 1, _i32)])
                c0 = plsc.load_gather(cf_v, [rows, jnp.zeros((16,), _i32)])
                c1 = plsc.load_gather(cf_v, [rows, jnp.full((16,), 1, _i32)])
                avv = av_v[pl.ds(g * 16, 16)]
                att = at_v[pl.ds(g * 16, 16)]
                a_v = jnp.exp(avv) / (sv + 1e-16)
                a_t = jnp.exp(att) / (st + 1e-16)
                wv = jnp.maximum(a_v * c0, a_t * c1)
                w_v[pl.ds(g * 16, 16)] = jnp.maximum(wv, 0.0)
            pltpu.sync_copy(w_v, w_hbm.at[pl.ds(cc, C)])
            return tok

        lax.fori_loop(c_beg, c_end, chunk_body, 0)

    return k(av, at, ms, conf16, srcs, dsts)


def _sage(x, srcs, dsts, wts, offs):
    @functools.partial(
        pl.kernel,
        out_type=jax.ShapeDtypeStruct((NPAD, 64), _f32),
        mesh=plsc.VectorSubcoreMesh(**_MESH),
        compiler_params=_CP,
        scratch_types=[
            pltpu.VMEM((64,), _i32),
            pltpu.VMEM((C,), _i32),
            pltpu.VMEM((C + 16,), _i32),
            pltpu.VMEM((C + 16,), _f32),
            pltpu.VMEM((C, 64), _f32),
            pltpu.VMEM((WX, 64), _f32),
            pltpu.SemaphoreType.DMA,
        ],
    )
    def k(x_hbm, src_hbm, dst_hbm, wt_hbm, off_hbm, out_hbm,
          off_v, src_v, dst_v, wt_v, rows_v, ow_v, sem):
        w = _wid()
        n0 = w * NPT_F
        n1 = n0 + NPT_F
        pltpu.sync_copy(off_hbm.at[pl.ds(0, 64)], off_v)
        e0 = _extract(off_v, w)
        e1 = _extract(off_v, w + 1)
        c_lo = e0 // C
        c_hi = (e1 + C - 1) // C

        zero16 = jnp.zeros((16,), _f32)

        def zero_out():
            def zb(r, t):
                for kk in range(4):
                    ow_v[r, pl.ds(kk * 16, 16)] = zero16
                return t
            lax.fori_loop(0, WX, zb, 0)

        def close_out(w0):
            @pl.when(w0 < n1)
            def _():
                wa = pl.multiple_of(w0, 8)
                pltpu.sync_copy(ow_v, out_hbm.at[pl.ds(wa, WX), pl.ds(0, 64)])
            zero_out()

        def flush(d_cur, wx0, a):
            r = d_cur - wx0
            for kk in range(4):
                ow_v[r, pl.ds(kk * 16, 16)] = _leaky_v(a[kk])

        zero_out()

        def chunk_body(c, carry):
            cc = pl.multiple_of(c * C, 8)
            pltpu.sync_copy(src_hbm.at[pl.ds(cc, C)], src_v)
            pltpu.sync_copy(dst_hbm.at[pl.ds(cc, C)], dst_v.at[pl.ds(0, C)])
            pltpu.sync_copy(wt_hbm.at[pl.ds(cc, C)], wt_v.at[pl.ds(0, C)])
            pltpu.async_copy(x_hbm.at[src_v], rows_v, sem).wait()
            j_lo = jnp.maximum(e0 - c * C, 0)
            j_hi = jnp.minimum(e1 - c * C, C)

            def edge_body(j, ecarry):
                d_cur, wx0, a0, a1, a2, a3 = ecarry
                d = _extract(dst_v, j)
                is_new = d != d_cur

                def do_flush(tok):
                    flush(d_cur, wx0, (a0, a1, a2, a3))
                    return tok
                lax.cond(is_new & (d_cur >= n0), do_flush, lambda t: t, 0)

                def adv(_, w0):
                    close_out(w0)
                    return w0 + WX
                n_adv = jnp.maximum(d - wx0, 0) // WX
                wx0 = lax.fori_loop(0, n_adv, adv, wx0)

                sel = jnp.full((16,), is_new)
                a0 = jnp.where(sel, zero16, a0)
                a1 = jnp.where(sel, zero16, a1)
                a2 = jnp.where(sel, zero16, a2)
                a3 = jnp.where(sel, zero16, a3)
                d_cur = d

                we = _bcast(wt_v[pl.ds(j, 16)][0])
                a0 = a0 + we * rows_v[j, pl.ds(0, 16)]
                a1 = a1 + we * rows_v[j, pl.ds(16, 16)]
                a2 = a2 + we * rows_v[j, pl.ds(32, 16)]
                a3 = a3 + we * rows_v[j, pl.ds(48, 16)]
                return (d_cur, wx0, a0, a1, a2, a3)

            return lax.fori_loop(j_lo, j_hi, edge_body, carry)

        init = (jnp.int32(-1), n0, zero16, zero16, zero16, zero16)
        carry = lax.fori_loop(c_lo, c_hi, chunk_body, init)
        d_cur, wx0, a0, a1, a2, a3 = carry

        def do_flush(tok):
            flush(d_cur, wx0, (a0, a1, a2, a3))
            return tok
        lax.cond(d_cur >= n0, do_flush, lambda t: t, 0)

        def drain(_, w0):
            close_out(w0)
            return w0 + WX
        lax.fori_loop(0, (n1 - wx0) // WX, drain, wx0)

    return k(x, srcs, dsts, wts, offs)


def _feats_tc(v_feat, t_feat, v_W, v_b, t_W, t_b):
    vf = jnp.pad(v_feat, ((0, 40960 - NUM_ITEM), (0, 0)))
    tf = jnp.pad(t_feat, ((0, 40960 - NUM_ITEM), (0, 0)))
    vb = jnp.broadcast_to(v_b[None, :], (8, 64))
    tb = jnp.broadcast_to(t_b[None, :], (8, 64))

    def body(vf_ref, tf_ref, vw_ref, tw_ref, vb_ref, tb_ref, o_ref):
        def half(f, wgt, b):
            y = jax.lax.dot_general(f, wgt, (((1,), (1,)), ((), ())),
                                    preferred_element_type=_f32)
            y = y + b[0:1, :]
            y = jnp.where(y >= 0, y, 0.01 * y)
            nrm = jnp.sqrt(jnp.sum(y * y, axis=1, keepdims=True))
            return y / jnp.maximum(nrm, 1e-12)
        o_ref[:, 0:64] = half(vf_ref[...], vw_ref[...], vb_ref[...])
        o_ref[:, 64:128] = half(tf_ref[...], tw_ref[...], tb_ref[...])

    return pl.pallas_call(
        body,
        grid=(40960 // 256,),
        in_specs=[
            pl.BlockSpec((256, 128), lambda i: (i, 0)),
            pl.BlockSpec((256, 128), lambda i: (i, 0)),
            pl.BlockSpec((64, 128), lambda i: (0, 0)),
            pl.BlockSpec((64, 128), lambda i: (0, 0)),
            pl.BlockSpec((8, 64), lambda i: (0, 0)),
            pl.BlockSpec((8, 64), lambda i: (0, 0)),
        ],
        out_specs=pl.BlockSpec((256, 128), lambda i: (i, 0)),
        out_shape=jax.ShapeDtypeStruct((40960, 128), _f32),
    )(vf, tf, v_W, t_W, vb, tb)


def _pref_update_tc(pref_cat, xhat):
    def body(p_ref, x_ref, o_ref):
        y = p_ref[...] + x_ref[...]
        for lo in (0, 64):
            h = y[:, lo:lo + 64]
            nrm = jnp.sqrt(jnp.sum(h * h, axis=1, keepdims=True))
            o_ref[:, lo:lo + 64] = h / jnp.maximum(nrm, 1e-12)

    return pl.pallas_call(
        body,
        grid=(UPAD // 256,),
        in_specs=[
            pl.BlockSpec((256, 128), lambda i: (i, 0)),
            pl.BlockSpec((256, 128), lambda i: (i, 0)),
        ],
        out_specs=pl.BlockSpec((256, 128), lambda i: (i, 0)),
        out_shape=jax.ShapeDtypeStruct((UPAD, 128), _f32),
    )(pref_cat, xhat)


def _l2norm_tc(x):
    def body(x_ref, o_ref):
        y = x_ref[...]
        nrm = jnp.sqrt(jnp.sum(y * y, axis=1, keepdims=True))
        o_ref[...] = y / jnp.maximum(nrm, 1e-12)

    return pl.pallas_call(
        body,
        grid=(NPAD // 256,),
        in_specs=[pl.BlockSpec((256, 64), lambda i: (i, 0))],
        out_specs=pl.BlockSpec((256, 64), lambda i: (i, 0)),
        out_shape=jax.ShapeDtypeStruct((NPAD, 64), _f32),
    )(x)


def _fuse_tc(x0, x1, x2, rep):
    def body(a_ref, b_ref, c_ref, r_ref, o_ref):
        o_ref[:, 0:64] = a_ref[...] + b_ref[...] + c_ref[...]
        o_ref[:, 64:192] = r_ref[...]

    return pl.pallas_call(
        body,
        grid=(NPAD // 256,),
        in_specs=[
            pl.BlockSpec((256, 64), lambda i: (i, 0)),
            pl.BlockSpec((256, 64), lambda i: (i, 0)),
            pl.BlockSpec((256, 64), lambda i: (i, 0)),
            pl.BlockSpec((256, 128), lambda i: (i, 0)),
        ],
        out_specs=pl.BlockSpec((256, 192), lambda i: (i, 0)),
        out_shape=jax.ShapeDtypeStruct((NPAD, 192), _f32),
    )(x0, x1, x2, rep)


def kernel(edge_index, v_feat, t_feat, v_pref, t_pref, v_W, v_b, t_W, t_b,
           id_emb, conf):
    src = edge_index[0].astype(_i32)
    dst = edge_index[1].astype(_i32)

    perm1 = jnp.argsort(dst)
    srcs1 = jnp.pad(src[perm1], (0, L1 - E)).astype(_i32)
    dsts1_u = dst[perm1]
    dsts1 = jnp.pad(dsts1_u, (0, L1 - E)).astype(_i32)
    b_r = jnp.minimum(jnp.arange(33) * NPT_R, NUM_USER)
    off_r = jnp.pad(jnp.searchsorted(dsts1_u, b_r).astype(_i32), (0, 31))

    src2 = jnp.concatenate([src, dst])
    dst2 = jnp.concatenate([dst, src])
    perm2 = jnp.argsort(dst2)
    srcs2 = jnp.pad(src2[perm2], (0, L2 - 2 * E)).astype(_i32)
    dsts2_u = dst2[perm2]
    dsts2 = jnp.pad(dsts2_u, (0, L2 - 2 * E)).astype(_i32)
    b_f = jnp.minimum(jnp.arange(33) * NPT_F, N)
    off_f = jnp.pad(jnp.searchsorted(dsts2_u, b_f).astype(_i32), (0, 31))

    feats_cat = _feats_tc(v_feat, t_feat, v_W, v_b, t_W, t_b)
    pref0 = jnp.pad(jnp.concatenate([v_pref, t_pref], axis=1),
                    ((0, UPAD - NUM_USER), (0, 0)))
    pref_cat = _pref_update_tc(pref0, jnp.zeros((UPAD, 128), _f32))

    pad_tail = jnp.zeros((NPAD - N, 128), _f32)

    for _ in range(NUM_ROUTING):
        x_cat = jnp.concatenate(
            [pref_cat[:NUM_USER], feats_cat[:NUM_ITEM], pad_tail])
        xh = _gat_routing(x_cat, srcs1, dsts1, off_r)
        pref_cat = _pref_update_tc(pref_cat, xh)

    x_cat = jnp.concatenate(
        [pref_cat[:NUM_USER], feats_cat[:NUM_ITEM], pad_tail])
    rep, ms, av, at = _gat_final(x_cat, srcs2, dsts2, off_f)

    conf16 = jnp.zeros((NPAD, 16), _f32).at[:N, 0:2].set(conf)
    wts = _edge_weights(av, at, ms, conf16, srcs2, dsts2)

    x0 = _l2norm_tc(jnp.pad(id_emb, ((0, NPAD - N), (0, 0))))
    x1 = _sage(x0, srcs2, dsts2, wts, off_f)
    x2 = _sage(x1, srcs2, dsts2, wts, off_f)

    out = _fuse_tc(x0, x1, x2, rep)
    return out[:N]


# speedup vs baseline: 13.5537x; 1.0145x over previous
import functools

import jax
import jax.numpy as jnp
from jax import lax
from jax.experimental import pallas as pl
from jax.experimental.pallas import tpu as pltpu
from jax.experimental.pallas import tpu_sc as plsc

NUM_USER = 10000
NUM_ITEM = 40000
N = NUM_USER + NUM_ITEM
E = 800000
NUM_ROUTING = 3

NW = 32
C = 256
WX = 64
NPT_F = 1600
NPAD = NPT_F * NW
NPT_R = 320
UPAD = NPT_R * NW
L1 = (E // C + 1) * C
L2 = (2 * E // C + 1) * C
NEG = -1e30

_MESH = dict(core_axis_name="c", subcore_axis_name="s", num_cores=2,
             num_subcores=16)
_CP = pltpu.CompilerParams(needs_layout_passes=False, use_tc_tiling_on_sc=False)

_f32 = jnp.float32
_i32 = jnp.int32


def _wid():
    return lax.axis_index("s") * 2 + lax.axis_index("c")


def _extract(ref, j):
    return ref[pl.ds(j, 16)][0]


def _bcast(x):
    return jnp.full((16,), x, _f32)


def _leaky_v(x):
    return jnp.where(x >= 0.0, x, 0.01 * x)


def _gat_routing(x_cat, srcs, dsts, offs):
    @functools.partial(
        pl.kernel,
        out_type=jax.ShapeDtypeStruct((UPAD, 128), _f32),
        mesh=plsc.VectorSubcoreMesh(**_MESH),
        compiler_params=_CP,
        scratch_types=[
            pltpu.VMEM((64,), _i32),
            pltpu.VMEM((C,), _i32),
            pltpu.VMEM((C + 16,), _i32),
            pltpu.VMEM((C, 128), _f32),
            pltpu.VMEM((WX, 128), _f32),
            pltpu.VMEM((WX, 128), _f32),
            pltpu.SemaphoreType.DMA,
        ],
    )
    def k(x_hbm, src_hbm, dst_hbm, off_hbm, out_hbm,
          off_v, src_v, dst_v, rows_v, xw_v, ow_v, sem):
        w = _wid()
        n0 = w * NPT_R
        n1 = n0 + NPT_R
        pltpu.sync_copy(off_hbm.at[pl.ds(0, 64)], off_v)
        e0 = _extract(off_v, w)
        e1 = _extract(off_v, w + 1)
        c_lo = e0 // C
        c_hi = (e1 + C - 1) // C

        zero16 = jnp.zeros((16,), _f32)

        def open_x(w0):
            wa = pl.multiple_of(w0, 8)
            pltpu.sync_copy(x_hbm.at[pl.ds(wa, WX), pl.ds(0, 128)], xw_v)

        def zero_out():
            def zb(r, t):
                for kk in range(8):
                    ow_v[r, pl.ds(kk * 16, 16)] = zero16
                return t
            lax.fori_loop(0, WX, zb, 0)

        def close_out(w0):
            @pl.when(w0 < n1)
            def _():
                wa = pl.multiple_of(w0, 8)
                pltpu.sync_copy(ow_v, out_hbm.at[pl.ds(wa, WX), pl.ds(0, 128)])
            zero_out()

        def flush(d_cur, wx0, s_v, s_t, a):
            r = d_cur - wx0
            inv_v = _bcast(1.0) / (s_v + 1e-16)
            inv_t = _bcast(1.0) / (s_t + 1e-16)
            for kk in range(4):
                ow_v[r, pl.ds(kk * 16, 16)] = a[kk] * inv_v
            for kk in range(4, 8):
                ow_v[r, pl.ds(kk * 16, 16)] = a[kk] * inv_t

        open_x(n0)
        zero_out()

        def chunk_body(c, carry):
            cc = pl.multiple_of(c * C, 8)
            pltpu.sync_copy(src_hbm.at[pl.ds(cc, C)], src_v)
            pltpu.sync_copy(dst_hbm.at[pl.ds(cc, C)], dst_v.at[pl.ds(0, C)])
            pltpu.async_copy(x_hbm.at[src_v], rows_v, sem).wait()
            j_lo = jnp.maximum(e0 - c * C, 0)
            j_hi = jnp.minimum(e1 - c * C, C)

            def edge_body(j, ecarry):
                (d_cur, wx0, s_v, s_t,
                 a0, a1, a2, a3, a4, a5, a6, a7) = ecarry
                d = _extract(dst_v, j)
                is_new = d != d_cur

                def do_new(w0):
                    @pl.when(d_cur >= n0)
                    def _():
                        flush(d_cur, w0, s_v, s_t,
                              (a0, a1, a2, a3, a4, a5, a6, a7))

                    def adv(_, wv):
                        close_out(wv)
                        wv = wv + WX
                        open_x(wv)
                        return wv
                    n_adv = jnp.maximum(d - w0, 0) // WX
                    return lax.fori_loop(0, n_adv, adv, w0)

                wx0 = lax.cond(is_new, do_new, lambda w0: w0, wx0)

                sel = jnp.full((16,), is_new)
                s_v = jnp.where(sel, zero16, s_v)
                s_t = jnp.where(sel, zero16, s_t)
                a0 = jnp.where(sel, zero16, a0)
                a1 = jnp.where(sel, zero16, a1)
                a2 = jnp.where(sel, zero16, a2)
                a3 = jnp.where(sel, zero16, a3)
                a4 = jnp.where(sel, zero16, a4)
                a5 = jnp.where(sel, zero16, a5)
                a6 = jnp.where(sel, zero16, a6)
                a7 = jnp.where(sel, zero16, a7)
                d_cur = d

                r = jnp.maximum(d - wx0, 0)
                r0 = rows_v[j, pl.ds(0, 16)]
                r1 = rows_v[j, pl.ds(16, 16)]
                r2 = rows_v[j, pl.ds(32, 16)]
                r3 = rows_v[j, pl.ds(48, 16)]
                r4 = rows_v[j, pl.ds(64, 16)]
                r5 = rows_v[j, pl.ds(80, 16)]
                r6 = rows_v[j, pl.ds(96, 16)]
                r7 = rows_v[j, pl.ds(112, 16)]
                dv = (r0 * xw_v[r, pl.ds(0, 16)] + r1 * xw_v[r, pl.ds(16, 16)]
                      + r2 * xw_v[r, pl.ds(32, 16)]
                      + r3 * xw_v[r, pl.ds(48, 16)])
                dt = (r4 * xw_v[r, pl.ds(64, 16)] + r5 * xw_v[r, pl.ds(80, 16)]
                      + r6 * xw_v[r, pl.ds(96, 16)]
                      + r7 * xw_v[r, pl.ds(112, 16)])
                valid = (j >= j_lo) & (j < j_hi)
                av = jnp.where(valid, jnp.sum(dv), NEG)
                at = jnp.where(valid, jnp.sum(dt), NEG)
                p_v = jnp.exp(_bcast(av))
                p_t = jnp.exp(_bcast(at))
                s_v = s_v + p_v
                s_t = s_t + p_t
                a0 = a0 + p_v * r0
                a1 = a1 + p_v * r1
                a2 = a2 + p_v * r2
                a3 = a3 + p_v * r3
                a4 = a4 + p_t * r4
                a5 = a5 + p_t * r5
                a6 = a6 + p_t * r6
                a7 = a7 + p_t * r7
                return (d_cur, wx0, s_v, s_t,
                        a0, a1, a2, a3, a4, a5, a6, a7)

            return lax.fori_loop(j_lo, j_hi, edge_body, carry)

        init = (jnp.int32(-1), n0,
                zero16, zero16,
                zero16, zero16, zero16, zero16,
                zero16, zero16, zero16, zero16)
        carry = lax.fori_loop(c_lo, c_hi, chunk_body, init)
        (d_cur, wx0, s_v, s_t,
         a0, a1, a2, a3, a4, a5, a6, a7) = carry

        def do_flush(tok):
            flush(d_cur, wx0, s_v, s_t, (a0, a1, a2, a3, a4, a5, a6, a7))
            return tok
        lax.cond(d_cur >= n0, do_flush, lambda t: t, 0)

        def drain(_, w0):
            close_out(w0)
            return w0 + WX
        lax.fori_loop(0, (n1 - wx0) // WX, drain, wx0)

    return k(x_cat, srcs, dsts, offs)


def _gat_final(x_cat, srcs, dsts, offs):
    @functools.partial(
        pl.kernel,
        out_type=(
            jax.ShapeDtypeStruct((NPAD, 128), _f32),
            jax.ShapeDtypeStruct((NPAD, 16), _f32),
            jax.ShapeDtypeStruct((L2,), _f32),
            jax.ShapeDtypeStruct((L2,), _f32),
        ),
        mesh=plsc.VectorSubcoreMesh(**_MESH),
        compiler_params=_CP,
        scratch_types=[
            pltpu.VMEM((64,), _i32),
            pltpu.VMEM((C,), _i32),
            pltpu.VMEM((C + 16,), _i32),
            pltpu.VMEM((C, 128), _f32),
            pltpu.VMEM((WX, 128), _f32),
            pltpu.VMEM((WX, 16), _f32),
            pltpu.VMEM((C,), _f32),
            pltpu.VMEM((C,), _f32),
            pltpu.SemaphoreType.DMA,
        ],
    )
    def k(x_hbm, src_hbm, dst_hbm, off_hbm,
          rep_hbm, ms_hbm, av_hbm, at_hbm,
          off_v, src_v, dst_v, rows_v, xw_v, msw_v, av_v, at_v, sem):
        w = _wid()
        n0 = w * NPT_F
        n1 = n0 + NPT_F
        pltpu.sync_copy(off_hbm.at[pl.ds(0, 64)], off_v)
        e0 = _extract(off_v, w)
        e1 = _extract(off_v, w + 1)
        c_lo = e0 // C
        c_hi = (e1 + C - 1) // C
        aligned0 = e0 == c_lo * C

        zero16 = jnp.zeros((16,), _f32)
        iota = lax.iota(_i32, 16)

        def open_x(w0):
            wa = pl.multiple_of(w0, 8)
            pltpu.sync_copy(x_hbm.at[pl.ds(wa, WX), pl.ds(0, 128)], xw_v)

        def zero_ms():
            def zb(r, t):
                msw_v[r, pl.ds(0, 16)] = zero16
                return t
            lax.fori_loop(0, WX, zb, 0)

        def close_win(w0):
            @pl.when(w0 < n1)
            def _():
                wa = pl.multiple_of(w0, 8)
                pltpu.sync_copy(xw_v, rep_hbm.at[pl.ds(wa, WX), pl.ds(0, 128)])
                pltpu.sync_copy(msw_v, ms_hbm.at[pl.ds(wa, WX), pl.ds(0, 16)])

        def flush(d_cur, wx0, s_v, s_t, a):
            r = d_cur - wx0
            inv_v = _bcast(1.0) / (s_v + 1e-16)
            inv_t = _bcast(1.0) / (s_t + 1e-16)
            for kk in range(4):
                x_row = xw_v[r, pl.ds(kk * 16, 16)]
                xw_v[r, pl.ds(kk * 16, 16)] = x_row + _leaky_v(a[kk] * inv_v)
            for kk in range(4, 8):
                x_row = xw_v[r, pl.ds(kk * 16, 16)]
                xw_v[r, pl.ds(kk * 16, 16)] = x_row + _leaky_v(a[kk] * inv_t)
            msrow = jnp.where(iota == 0, s_v, jnp.where(iota == 1, s_t, zero16))
            msw_v[r, pl.ds(0, 16)] = msrow

        open_x(n0)
        zero_ms()

        def chunk_body(c, carry):
            cc = pl.multiple_of(c * C, 8)
            pltpu.sync_copy(src_hbm.at[pl.ds(cc, C)], src_v)
            pltpu.sync_copy(dst_hbm.at[pl.ds(cc, C)], dst_v.at[pl.ds(0, C)])
            pltpu.async_copy(x_hbm.at[src_v], rows_v, sem).wait()
            j_lo = jnp.maximum(e0 - c * C, 0)
            j_hi = jnp.minimum(e1 - c * C, C)

            def edge_body(j, ecarry):
                (d_cur, wx0, s_v, s_t,
                 a0, a1, a2, a3, a4, a5, a6, a7, av_acc, at_acc) = ecarry
                d = _extract(dst_v, j)
                is_new = d != d_cur

                def do_new(w0):
                    @pl.when((d_cur >= n0) & (d_cur < n1))
                    def _():
                        flush(d_cur, w0, s_v, s_t,
                              (a0, a1, a2, a3, a4, a5, a6, a7))

                    def adv(_, wv):
                        close_win(wv)
                        wv = wv + WX
                        open_x(wv)
                        zero_ms()
                        return wv
                    n_adv = jnp.maximum(d - w0, 0) // WX
                    return lax.fori_loop(0, n_adv, adv, w0)

                wx0 = lax.cond(is_new, do_new, lambda w0: w0, wx0)

                sel = jnp.full((16,), is_new)
                s_v = jnp.where(sel, zero16, s_v)
                s_t = jnp.where(sel, zero16, s_t)
                a0 = jnp.where(sel, zero16, a0)
                a1 = jnp.where(sel, zero16, a1)
                a2 = jnp.where(sel, zero16, a2)
                a3 = jnp.where(sel, zero16, a3)
                a4 = jnp.where(sel, zero16, a4)
                a5 = jnp.where(sel, zero16, a5)
                a6 = jnp.where(sel, zero16, a6)
                a7 = jnp.where(sel, zero16, a7)
                d_cur = d

                r = jnp.maximum(d - wx0, 0)
                r0 = rows_v[j, pl.ds(0, 16)]
                r1 = rows_v[j, pl.ds(16, 16)]
                r2 = rows_v[j, pl.ds(32, 16)]
                r3 = rows_v[j, pl.ds(48, 16)]
                r4 = rows_v[j, pl.ds(64, 16)]
                r5 = rows_v[j, pl.ds(80, 16)]
                r6 = rows_v[j, pl.ds(96, 16)]
                r7 = rows_v[j, pl.ds(112, 16)]
                dv = (r0 * xw_v[r, pl.ds(0, 16)] + r1 * xw_v[r, pl.ds(16, 16)]
                      + r2 * xw_v[r, pl.ds(32, 16)]
                      + r3 * xw_v[r, pl.ds(48, 16)])
                dt = (r4 * xw_v[r, pl.ds(64, 16)] + r5 * xw_v[r, pl.ds(80, 16)]
                      + r6 * xw_v[r, pl.ds(96, 16)]
                      + r7 * xw_v[r, pl.ds(112, 16)])
                alpha_v = jnp.sum(dv)
                alpha_t = jnp.sum(dt)

                lane = j - (j // 16) * 16
                lane_sel = iota == lane
                av_acc = jnp.where(lane_sel, _bcast(alpha_v), av_acc)
                at_acc = jnp.where(lane_sel, _bcast(alpha_t), at_acc)

                @pl.when(lane == 15)
                def _():
                    base = pl.multiple_of(j - 15, 16)
                    av_v[pl.ds(base, 16)] = av_acc
                    at_v[pl.ds(base, 16)] = at_acc

                valid = (j >= j_lo) & (j < j_hi)
                av = jnp.where(valid, alpha_v, NEG)
                at = jnp.where(valid, alpha_t, NEG)
                p_v = jnp.exp(_bcast(av))
                p_t = jnp.exp(_bcast(at))
                s_v = s_v + p_v
                s_t = s_t + p_t
                a0 = a0 + p_v * r0
                a1 = a1 + p_v * r1
                a2 = a2 + p_v * r2
                a3 = a3 + p_v * r3
                a4 = a4 + p_t * r4
                a5 = a5 + p_t * r5
                a6 = a6 + p_t * r6
                a7 = a7 + p_t * r7
                return (d_cur, wx0, s_v, s_t,
                        a0, a1, a2, a3, a4, a5, a6, a7, av_acc, at_acc)

            carry = lax.fori_loop(j_lo, C, edge_body, carry)

            @pl.when((c > c_lo) | aligned0)
            def _():
                pltpu.sync_copy(av_v, av_hbm.at[pl.ds(cc, C)])
                pltpu.sync_copy(at_v, at_hbm.at[pl.ds(cc, C)])
            return carry

        init = (jnp.int32(-1), n0,
                zero16, zero16,
                zero16, zero16, zero16, zero16,
                zero16, zero16, zero16, zero16,
                zero16, zero16)
        carry = lax.fori_loop(c_lo, c_hi, chunk_body, init)
        (d_cur, wx0, s_v, s_t,
         a0, a1, a2, a3, a4, a5, a6, a7, _av, _at) = carry

        def do_flush(tok):
            flush(d_cur, wx0, s_v, s_t,
                  (a0, a1, a2, a3, a4, a5, a6, a7))
            return tok
        lax.cond((d_cur >= n0) & (d_cur < n1), do_flush, lambda t: t, 0)

        def drain(_, w0):
            close_win(w0)
            w0 = w0 + WX
            @pl.when(w0 < n1)
            def _():
                open_x(w0)
                zero_ms()
            return w0
        lax.fori_loop(0, (n1 - wx0) // WX, drain, wx0)

    return k(x_cat, srcs, dsts, offs)


def _edge_weights(av, at, ms, conf16, srcs, dsts):
    total_chunks = L2 // C
    kpt = (total_chunks + NW - 1) // NW

    @functools.partial(
        pl.kernel,
        out_type=jax.ShapeDtypeStruct((L2,), _f32),
        mesh=plsc.VectorSubcoreMesh(**_MESH),
        compiler_params=_CP,
        scratch_types=[
            pltpu.VMEM((C,), _i32),
            pltpu.VMEM((C,), _i32),
            pltpu.VMEM((C,), _f32),
            pltpu.VMEM((C,), _f32),
            pltpu.VMEM((C, 16), _f32),
            pltpu.VMEM((C, 16), _f32),
            pltpu.VMEM((C,), _f32),
            pltpu.SemaphoreType.DMA,
            pltpu.SemaphoreType.DMA,
        ],
    )
    def k(av_hbm, at_hbm, ms_hbm, cf_hbm, src_hbm, dst_hbm, w_hbm,
          src_v, dst_v, av_v, at_v, ms_v, cf_v, w_v, sem1, sem2):
        w = _wid()
        c_beg = w * kpt
        c_end = jnp.minimum(c_beg + kpt, total_chunks)
        iota = lax.iota(_i32, 16)

        def chunk_body(c, tok):
            cc = pl.multiple_of(c * C, 8)
            pltpu.sync_copy(src_hbm.at[pl.ds(cc, C)], src_v)
            pltpu.sync_copy(dst_hbm.at[pl.ds(cc, C)], dst_v)
            pltpu.sync_copy(av_hbm.at[pl.ds(cc, C)], av_v)
            pltpu.sync_copy(at_hbm.at[pl.ds(cc, C)], at_v)
            cp1 = pltpu.async_copy(ms_hbm.at[dst_v], ms_v, sem1)
            cp2 = pltpu.async_copy(cf_hbm.at[src_v], cf_v, sem2)
            cp1.wait()
            cp2.wait()
            for g in range(C // 16):
                rows = iota + g * 16
                sv = plsc.load_gather(ms_v, [rows, jnp.zeros((16,), _i32)])
                st = plsc.load_gather(ms_v, [rows, jnp.full((16,), 1, _i32)])
                c0 = plsc.load_gather(cf_v, [rows, jnp.zeros((16,), _i32)])
                c1 = plsc.load_gather(cf_v, [rows, jnp.full((16,), 1, _i32)])
                avv = av_v[pl.ds(g * 16, 16)]
                att = at_v[pl.ds(g * 16, 16)]
                a_v = jnp.exp(avv) / (sv + 1e-16)
                a_t = jnp.exp(att) / (st + 1e-16)
                wv = jnp.maximum(a_v * c0, a_t * c1)
                w_v[pl.ds(g * 16, 16)] = jnp.maximum(wv, 0.0)
            pltpu.sync_copy(w_v, w_hbm.at[pl.ds(cc, C)])
            return tok

        lax.fori_loop(c_beg, c_end, chunk_body, 0)

    return k(av, at, ms, conf16, srcs, dsts)


def _sage(x, srcs, dsts, wts, offs):
    @functools.partial(
        pl.kernel,
        out_type=jax.ShapeDtypeStruct((NPAD, 64), _f32),
        mesh=plsc.VectorSubcoreMesh(**_MESH),
        compiler_params=_CP,
        scratch_types=[
            pltpu.VMEM((64,), _i32),
            pltpu.VMEM((C,), _i32),
            pltpu.VMEM((C + 16,), _i32),
            pltpu.VMEM((C + 16,), _f32),
            pltpu.VMEM((C, 64), _f32),
            pltpu.VMEM((WX, 64), _f32),
            pltpu.SemaphoreType.DMA,
        ],
    )
    def k(x_hbm, src_hbm, dst_hbm, wt_hbm, off_hbm, out_hbm,
          off_v, src_v, dst_v, wt_v, rows_v, ow_v, sem):
        w = _wid()
        n0 = w * NPT_F
        n1 = n0 + NPT_F
        pltpu.sync_copy(off_hbm.at[pl.ds(0, 64)], off_v)
        e0 = _extract(off_v, w)
        e1 = _extract(off_v, w + 1)
        c_lo = e0 // C
        c_hi = (e1 + C - 1) // C

        zero16 = jnp.zeros((16,), _f32)

        def zero_out():
            def zb(r, t):
                for kk in range(4):
                    ow_v[r, pl.ds(kk * 16, 16)] = zero16
                return t
            lax.fori_loop(0, WX, zb, 0)

        def close_out(w0):
            @pl.when(w0 < n1)
            def _():
                wa = pl.multiple_of(w0, 8)
                pltpu.sync_copy(ow_v, out_hbm.at[pl.ds(wa, WX), pl.ds(0, 64)])
            zero_out()

        def flush(d_cur, wx0, a):
            r = d_cur - wx0
            for kk in range(4):
                ow_v[r, pl.ds(kk * 16, 16)] = _leaky_v(a[kk])

        zero_out()

        def chunk_body(c, carry):
            cc = pl.multiple_of(c * C, 8)
            pltpu.sync_copy(src_hbm.at[pl.ds(cc, C)], src_v)
            pltpu.sync_copy(dst_hbm.at[pl.ds(cc, C)], dst_v.at[pl.ds(0, C)])
            pltpu.sync_copy(wt_hbm.at[pl.ds(cc, C)], wt_v.at[pl.ds(0, C)])
            pltpu.async_copy(x_hbm.at[src_v], rows_v, sem).wait()
            j_lo = jnp.maximum(e0 - c * C, 0)
            j_hi = jnp.minimum(e1 - c * C, C)

            def edge_body(j, ecarry):
                d_cur, wx0, a0, a1, a2, a3 = ecarry
                d = _extract(dst_v, j)
                is_new = d != d_cur

                def do_new(w0):
                    @pl.when(d_cur >= n0)
                    def _():
                        flush(d_cur, w0, (a0, a1, a2, a3))

                    def adv(_, wv):
                        close_out(wv)
                        return wv + WX
                    n_adv = jnp.maximum(d - w0, 0) // WX
                    return lax.fori_loop(0, n_adv, adv, w0)

                wx0 = lax.cond(is_new, do_new, lambda w0: w0, wx0)

                sel = jnp.full((16,), is_new)
                a0 = jnp.where(sel, zero16, a0)
                a1 = jnp.where(sel, zero16, a1)
                a2 = jnp.where(sel, zero16, a2)
                a3 = jnp.where(sel, zero16, a3)
                d_cur = d

                we = _bcast(wt_v[pl.ds(j, 16)][0])
                a0 = a0 + we * rows_v[j, pl.ds(0, 16)]
                a1 = a1 + we * rows_v[j, pl.ds(16, 16)]
                a2 = a2 + we * rows_v[j, pl.ds(32, 16)]
                a3 = a3 + we * rows_v[j, pl.ds(48, 16)]
                return (d_cur, wx0, a0, a1, a2, a3)

            return lax.fori_loop(j_lo, j_hi, edge_body, carry)

        init = (jnp.int32(-1), n0, zero16, zero16, zero16, zero16)
        carry = lax.fori_loop(c_lo, c_hi, chunk_body, init)
        d_cur, wx0, a0, a1, a2, a3 = carry

        def do_flush(tok):
            flush(d_cur, wx0, (a0, a1, a2, a3))
            return tok
        lax.cond(d_cur >= n0, do_flush, lambda t: t, 0)

        def drain(_, w0):
            close_out(w0)
            return w0 + WX
        lax.fori_loop(0, (n1 - wx0) // WX, drain, wx0)

    return k(x, srcs, dsts, wts, offs)


def _feats_tc(v_feat, t_feat, v_W, v_b, t_W, t_b):
    vf = jnp.pad(v_feat, ((0, 40960 - NUM_ITEM), (0, 0)))
    tf = jnp.pad(t_feat, ((0, 40960 - NUM_ITEM), (0, 0)))
    vb = jnp.broadcast_to(v_b[None, :], (8, 64))
    tb = jnp.broadcast_to(t_b[None, :], (8, 64))

    def body(vf_ref, tf_ref, vw_ref, tw_ref, vb_ref, tb_ref, o_ref):
        def half(f, wgt, b):
            y = jax.lax.dot_general(f, wgt, (((1,), (1,)), ((), ())),
                                    preferred_element_type=_f32)
            y = y + b[0:1, :]
            y = jnp.where(y >= 0, y, 0.01 * y)
            nrm = jnp.sqrt(jnp.sum(y * y, axis=1, keepdims=True))
            return y / jnp.maximum(nrm, 1e-12)
        o_ref[:, 0:64] = half(vf_ref[...], vw_ref[...], vb_ref[...])
        o_ref[:, 64:128] = half(tf_ref[...], tw_ref[...], tb_ref[...])

    return pl.pallas_call(
        body,
        grid=(40960 // 256,),
        in_specs=[
            pl.BlockSpec((256, 128), lambda i: (i, 0)),
            pl.BlockSpec((256, 128), lambda i: (i, 0)),
            pl.BlockSpec((64, 128), lambda i: (0, 0)),
            pl.BlockSpec((64, 128), lambda i: (0, 0)),
            pl.BlockSpec((8, 64), lambda i: (0, 0)),
            pl.BlockSpec((8, 64), lambda i: (0, 0)),
        ],
        out_specs=pl.BlockSpec((256, 128), lambda i: (i, 0)),
        out_shape=jax.ShapeDtypeStruct((40960, 128), _f32),
    )(vf, tf, v_W, t_W, vb, tb)


def _pref_update_tc(pref_cat, xhat):
    def body(p_ref, x_ref, o_ref):
        y = p_ref[...] + x_ref[...]
        for lo in (0, 64):
            h = y[:, lo:lo + 64]
            nrm = jnp.sqrt(jnp.sum(h * h, axis=1, keepdims=True))
            o_ref[:, lo:lo + 64] = h / jnp.maximum(nrm, 1e-12)

    return pl.pallas_call(
        body,
        grid=(UPAD // 256,),
        in_specs=[
            pl.BlockSpec((256, 128), lambda i: (i, 0)),
            pl.BlockSpec((256, 128), lambda i: (i, 0)),
        ],
        out_specs=pl.BlockSpec((256, 128), lambda i: (i, 0)),
        out_shape=jax.ShapeDtypeStruct((UPAD, 128), _f32),
    )(pref_cat, xhat)


def _l2norm_tc(x):
    def body(x_ref, o_ref):
        y = x_ref[...]
        nrm = jnp.sqrt(jnp.sum(y * y, axis=1, keepdims=True))
        o_ref[...] = y / jnp.maximum(nrm, 1e-12)

    return pl.pallas_call(
        body,
        grid=(NPAD // 256,),
        in_specs=[pl.BlockSpec((256, 64), lambda i: (i, 0))],
        out_specs=pl.BlockSpec((256, 64), lambda i: (i, 0)),
        out_shape=jax.ShapeDtypeStruct((NPAD, 64), _f32),
    )(x)


def _fuse_tc(x0, x1, x2, rep):
    def body(a_ref, b_ref, c_ref, r_ref, o_ref):
        o_ref[:, 0:64] = a_ref[...] + b_ref[...] + c_ref[...]
        o_ref[:, 64:192] = r_ref[...]

    return pl.pallas_call(
        body,
        grid=(NPAD // 256,),
        in_specs=[
            pl.BlockSpec((256, 64), lambda i: (i, 0)),
            pl.BlockSpec((256, 64), lambda i: (i, 0)),
            pl.BlockSpec((256, 64), lambda i: (i, 0)),
            pl.BlockSpec((256, 128), lambda i: (i, 0)),
        ],
        out_specs=pl.BlockSpec((256, 192), lambda i: (i, 0)),
        out_shape=jax.ShapeDtypeStruct((NPAD, 192), _f32),
    )(x0, x1, x2, rep)


def kernel(edge_index, v_feat, t_feat, v_pref, t_pref, v_W, v_b, t_W, t_b,
           id_emb, conf):
    src = edge_index[0].astype(_i32)
    dst = edge_index[1].astype(_i32)

    perm1 = jnp.argsort(dst)
    srcs1 = jnp.pad(src[perm1], (0, L1 - E)).astype(_i32)
    dsts1_u = dst[perm1]
    dsts1 = jnp.pad(dsts1_u, (0, L1 - E)).astype(_i32)
    b_r = jnp.minimum(jnp.arange(33) * NPT_R, NUM_USER)
    off_r = jnp.pad(jnp.searchsorted(dsts1_u, b_r).astype(_i32), (0, 31))

    src2 = jnp.concatenate([src, dst])
    dst2 = jnp.concatenate([dst, src])
    perm2 = jnp.argsort(dst2)
    srcs2 = jnp.pad(src2[perm2], (0, L2 - 2 * E)).astype(_i32)
    dsts2_u = dst2[perm2]
    dsts2 = jnp.pad(dsts2_u, (0, L2 - 2 * E)).astype(_i32)
    b_f = jnp.minimum(jnp.arange(33) * NPT_F, N)
    off_f = jnp.pad(jnp.searchsorted(dsts2_u, b_f).astype(_i32), (0, 31))

    feats_cat = _feats_tc(v_feat, t_feat, v_W, v_b, t_W, t_b)
    pref0 = jnp.pad(jnp.concatenate([v_pref, t_pref], axis=1),
                    ((0, UPAD - NUM_USER), (0, 0)))
    pref_cat = _pref_update_tc(pref0, jnp.zeros((UPAD, 128), _f32))

    pad_tail = jnp.zeros((NPAD - N, 128), _f32)

    for _ in range(NUM_ROUTING):
        x_cat = jnp.concatenate(
            [pref_cat[:NUM_USER], feats_cat[:NUM_ITEM], pad_tail])
        xh = _gat_routing(x_cat, srcs1, dsts1, off_r)
        pref_cat = _pref_update_tc(pref_cat, xh)

    x_cat = jnp.concatenate(
        [pref_cat[:NUM_USER], feats_cat[:NUM_ITEM], pad_tail])
    rep, ms, av, at = _gat_final(x_cat, srcs2, dsts2, off_f)

    conf16 = jnp.zeros((NPAD, 16), _f32).at[:N, 0:2].set(conf)
    wts = _edge_weights(av, at, ms, conf16, srcs2, dsts2)

    x0 = _l2norm_tc(jnp.pad(id_emb, ((0, NPAD - N), (0, 0))))
    x1 = _sage(x0, srcs2, dsts2, wts, off_f)
    x2 = _sage(x1, srcs2, dsts2, wts, off_f)

    out = _fuse_tc(x0, x1, x2, rep)
    return out[:N]


# speedup vs baseline: 14.0566x; 1.0371x over previous
import functools

import jax
import jax.numpy as jnp
from jax import lax
from jax.experimental import pallas as pl
from jax.experimental.pallas import tpu as pltpu
from jax.experimental.pallas import tpu_sc as plsc

NUM_USER = 10000
NUM_ITEM = 40000
N = NUM_USER + NUM_ITEM
E = 800000
NUM_ROUTING = 3

NW = 32
C = 256
WX = 64
NPT_F = 1600
NPAD = NPT_F * NW
NPT_R = 320
UPAD = NPT_R * NW
L1 = (E // C + 1) * C
L2 = (2 * E // C + 1) * C
NEG = -1e30

_MESH = dict(core_axis_name="c", subcore_axis_name="s", num_cores=2,
             num_subcores=16)
_CP = pltpu.CompilerParams(needs_layout_passes=False, use_tc_tiling_on_sc=False)

_f32 = jnp.float32
_i32 = jnp.int32


def _wid():
    return lax.axis_index("s") * 2 + lax.axis_index("c")


def _extract(ref, j):
    return ref[pl.ds(j, 16)][0]


def _bcast(x):
    return jnp.full((16,), x, _f32)


def _leaky_v(x):
    return jnp.where(x >= 0.0, x, 0.01 * x)


def _gat_routing(x_cat, srcs, dsts, offs):
    @functools.partial(
        pl.kernel,
        out_type=jax.ShapeDtypeStruct((UPAD, 128), _f32),
        mesh=plsc.VectorSubcoreMesh(**_MESH),
        compiler_params=_CP,
        scratch_types=[
            pltpu.VMEM((64,), _i32),
            pltpu.VMEM((2 * C,), _i32),
            pltpu.VMEM((2 * C + 16,), _i32),
            pltpu.VMEM((2 * C, 128), _f32),
            pltpu.VMEM((WX, 128), _f32),
            pltpu.VMEM((WX, 128), _f32),
            pltpu.SemaphoreType.DMA,
            pltpu.SemaphoreType.DMA,
        ],
    )
    def k(x_hbm, src_hbm, dst_hbm, off_hbm, out_hbm,
          off_v, src_v, dst_v, rows_v, xw_v, ow_v, sem_a, sem_b):
        w = _wid()
        n0 = w * NPT_R
        n1 = n0 + NPT_R
        pltpu.sync_copy(off_hbm.at[pl.ds(0, 64)], off_v)
        e0 = _extract(off_v, w)
        e1 = _extract(off_v, w + 1)
        c_lo = e0 // C
        c_hi = (e1 + C - 1) // C

        zero16 = jnp.zeros((16,), _f32)

        def open_x(w0):
            wa = pl.multiple_of(w0, 8)
            pltpu.sync_copy(x_hbm.at[pl.ds(wa, WX), pl.ds(0, 128)], xw_v)

        def zero_out():
            def zb(r, t):
                for kk in range(8):
                    ow_v[r, pl.ds(kk * 16, 16)] = zero16
                return t
            lax.fori_loop(0, WX, zb, 0)

        def close_out(w0):
            @pl.when(w0 < n1)
            def _():
                wa = pl.multiple_of(w0, 8)
                pltpu.sync_copy(ow_v, out_hbm.at[pl.ds(wa, WX), pl.ds(0, 128)])
            zero_out()

        def flush(d_cur, wx0, s_v, s_t, a):
            r = d_cur - wx0
            inv_v = _bcast(1.0) / (s_v + 1e-16)
            inv_t = _bcast(1.0) / (s_t + 1e-16)
            for kk in range(4):
                ow_v[r, pl.ds(kk * 16, 16)] = a[kk] * inv_v
            for kk in range(4, 8):
                ow_v[r, pl.ds(kk * 16, 16)] = a[kk] * inv_t

        def start_chunk(c, par):
            cc = pl.multiple_of(c * C, 8)

            @pl.when(par == 0)
            def _():
                pltpu.sync_copy(src_hbm.at[pl.ds(cc, C)], src_v.at[pl.ds(0, C)])
                pltpu.sync_copy(dst_hbm.at[pl.ds(cc, C)], dst_v.at[pl.ds(0, C)])
                pltpu.async_copy(x_hbm.at[src_v.at[pl.ds(0, C)]],
                                 rows_v.at[pl.ds(0, C)], sem_a)

            @pl.when(par == 1)
            def _():
                pltpu.sync_copy(src_hbm.at[pl.ds(cc, C)], src_v.at[pl.ds(C, C)])
                pltpu.sync_copy(dst_hbm.at[pl.ds(cc, C)], dst_v.at[pl.ds(C, C)])
                pltpu.async_copy(x_hbm.at[src_v.at[pl.ds(C, C)]],
                                 rows_v.at[pl.ds(C, C)], sem_b)

        def wait_rows(par):
            @pl.when(par == 0)
            def _():
                pltpu.make_async_copy(x_hbm.at[src_v.at[pl.ds(0, C)]],
                                      rows_v.at[pl.ds(0, C)], sem_a).wait()

            @pl.when(par == 1)
            def _():
                pltpu.make_async_copy(x_hbm.at[src_v.at[pl.ds(C, C)]],
                                      rows_v.at[pl.ds(C, C)], sem_b).wait()

        open_x(n0)
        zero_out()

        @pl.when(c_lo < c_hi)
        def _():
            start_chunk(c_lo, 0)

        def chunk_body(c, carry):
            par = lax.rem(c - c_lo, 2)
            base = par * C

            @pl.when(c + 1 < c_hi)
            def _():
                start_chunk(c + 1, 1 - par)

            wait_rows(par)
            j_lo = jnp.maximum(e0 - c * C, 0)
            j_hi = jnp.minimum(e1 - c * C, C)

            def edge_body(j, ecarry):
                (d_cur, wx0, s_v, s_t,
                 a0, a1, a2, a3, a4, a5, a6, a7) = ecarry
                jb = base + j
                d = _extract(dst_v, jb)
                is_new = d != d_cur

                def do_new(w0):
                    @pl.when(d_cur >= n0)
                    def _():
                        flush(d_cur, w0, s_v, s_t,
                              (a0, a1, a2, a3, a4, a5, a6, a7))

                    def adv(_, wv):
                        close_out(wv)
                        wv = wv + WX
                        open_x(wv)
                        return wv
                    n_adv = jnp.maximum(d - w0, 0) // WX
                    return lax.fori_loop(0, n_adv, adv, w0)

                wx0 = lax.cond(is_new, do_new, lambda w0: w0, wx0)

                sel = jnp.full((16,), is_new)
                s_v = jnp.where(sel, zero16, s_v)
                s_t = jnp.where(sel, zero16, s_t)
                a0 = jnp.where(sel, zero16, a0)
                a1 = jnp.where(sel, zero16, a1)
                a2 = jnp.where(sel, zero16, a2)
                a3 = jnp.where(sel, zero16, a3)
                a4 = jnp.where(sel, zero16, a4)
                a5 = jnp.where(sel, zero16, a5)
                a6 = jnp.where(sel, zero16, a6)
                a7 = jnp.where(sel, zero16, a7)
                d_cur = d

                r = jnp.maximum(d - wx0, 0)
                r0 = rows_v[jb, pl.ds(0, 16)]
                r1 = rows_v[jb, pl.ds(16, 16)]
                r2 = rows_v[jb, pl.ds(32, 16)]
                r3 = rows_v[jb, pl.ds(48, 16)]
                r4 = rows_v[jb, pl.ds(64, 16)]
                r5 = rows_v[jb, pl.ds(80, 16)]
                r6 = rows_v[jb, pl.ds(96, 16)]
                r7 = rows_v[jb, pl.ds(112, 16)]
                dv = (r0 * xw_v[r, pl.ds(0, 16)] + r1 * xw_v[r, pl.ds(16, 16)]
                      + r2 * xw_v[r, pl.ds(32, 16)]
                      + r3 * xw_v[r, pl.ds(48, 16)])
                dt = (r4 * xw_v[r, pl.ds(64, 16)] + r5 * xw_v[r, pl.ds(80, 16)]
                      + r6 * xw_v[r, pl.ds(96, 16)]
                      + r7 * xw_v[r, pl.ds(112, 16)])
                valid = (j >= j_lo) & (j < j_hi)
                av = jnp.where(valid, jnp.sum(dv), NEG)
                at = jnp.where(valid, jnp.sum(dt), NEG)
                p_v = jnp.exp(_bcast(av))
                p_t = jnp.exp(_bcast(at))
                s_v = s_v + p_v
                s_t = s_t + p_t
                a0 = a0 + p_v * r0
                a1 = a1 + p_v * r1
                a2 = a2 + p_v * r2
                a3 = a3 + p_v * r3
                a4 = a4 + p_t * r4
                a5 = a5 + p_t * r5
                a6 = a6 + p_t * r6
                a7 = a7 + p_t * r7
                return (d_cur, wx0, s_v, s_t,
                        a0, a1, a2, a3, a4, a5, a6, a7)

            return lax.fori_loop(j_lo, j_hi, edge_body, carry)

        init = (jnp.int32(-1), n0,
                zero16, zero16,
                zero16, zero16, zero16, zero16,
                zero16, zero16, zero16, zero16)
        carry = lax.fori_loop(c_lo, c_hi, chunk_body, init)
        (d_cur, wx0, s_v, s_t,
         a0, a1, a2, a3, a4, a5, a6, a7) = carry

        def do_flush(tok):
            flush(d_cur, wx0, s_v, s_t, (a0, a1, a2, a3, a4, a5, a6, a7))
            return tok
        lax.cond(d_cur >= n0, do_flush, lambda t: t, 0)

        def drain(_, w0):
            close_out(w0)
            return w0 + WX
        lax.fori_loop(0, (n1 - wx0) // WX, drain, wx0)

    return k(x_cat, srcs, dsts, offs)


def _gat_final(x_cat, srcs, dsts, offs):
    @functools.partial(
        pl.kernel,
        out_type=(
            jax.ShapeDtypeStruct((NPAD, 128), _f32),
            jax.ShapeDtypeStruct((NPAD, 16), _f32),
            jax.ShapeDtypeStruct((L2,), _f32),
            jax.ShapeDtypeStruct((L2,), _f32),
        ),
        mesh=plsc.VectorSubcoreMesh(**_MESH),
        compiler_params=_CP,
        scratch_types=[
            pltpu.VMEM((64,), _i32),
            pltpu.VMEM((2 * C,), _i32),
            pltpu.VMEM((2 * C + 16,), _i32),
            pltpu.VMEM((2 * C, 128), _f32),
            pltpu.VMEM((WX, 128), _f32),
            pltpu.VMEM((WX, 16), _f32),
            pltpu.VMEM((C,), _f32),
            pltpu.VMEM((C,), _f32),
            pltpu.SemaphoreType.DMA,
            pltpu.SemaphoreType.DMA,
        ],
    )
    def k(x_hbm, src_hbm, dst_hbm, off_hbm,
          rep_hbm, ms_hbm, av_hbm, at_hbm,
          off_v, src_v, dst_v, rows_v, xw_v, msw_v, av_v, at_v, sem_a, sem_b):
        w = _wid()
        n0 = w * NPT_F
        n1 = n0 + NPT_F
        pltpu.sync_copy(off_hbm.at[pl.ds(0, 64)], off_v)
        e0 = _extract(off_v, w)
        e1 = _extract(off_v, w + 1)
        c_lo = e0 // C
        c_hi = (e1 + C - 1) // C
        aligned0 = e0 == c_lo * C

        zero16 = jnp.zeros((16,), _f32)
        iota = lax.iota(_i32, 16)

        def open_x(w0):
            wa = pl.multiple_of(w0, 8)
            pltpu.sync_copy(x_hbm.at[pl.ds(wa, WX), pl.ds(0, 128)], xw_v)

        def zero_ms():
            def zb(r, t):
                msw_v[r, pl.ds(0, 16)] = zero16
                return t
            lax.fori_loop(0, WX, zb, 0)

        def close_win(w0):
            @pl.when(w0 < n1)
            def _():
                wa = pl.multiple_of(w0, 8)
                pltpu.sync_copy(xw_v, rep_hbm.at[pl.ds(wa, WX), pl.ds(0, 128)])
                pltpu.sync_copy(msw_v, ms_hbm.at[pl.ds(wa, WX), pl.ds(0, 16)])

        def flush(d_cur, wx0, s_v, s_t, a):
            r = d_cur - wx0
            inv_v = _bcast(1.0) / (s_v + 1e-16)
            inv_t = _bcast(1.0) / (s_t + 1e-16)
            for kk in range(4):
                x_row = xw_v[r, pl.ds(kk * 16, 16)]
                xw_v[r, pl.ds(kk * 16, 16)] = x_row + _leaky_v(a[kk] * inv_v)
            for kk in range(4, 8):
                x_row = xw_v[r, pl.ds(kk * 16, 16)]
                xw_v[r, pl.ds(kk * 16, 16)] = x_row + _leaky_v(a[kk] * inv_t)
            msrow = jnp.where(iota == 0, s_v, jnp.where(iota == 1, s_t, zero16))
            msw_v[r, pl.ds(0, 16)] = msrow

        def start_chunk(c, par):
            cc = pl.multiple_of(c * C, 8)

            @pl.when(par == 0)
            def _():
                pltpu.sync_copy(src_hbm.at[pl.ds(cc, C)], src_v.at[pl.ds(0, C)])
                pltpu.sync_copy(dst_hbm.at[pl.ds(cc, C)], dst_v.at[pl.ds(0, C)])
                pltpu.async_copy(x_hbm.at[src_v.at[pl.ds(0, C)]],
                                 rows_v.at[pl.ds(0, C)], sem_a)

            @pl.when(par == 1)
            def _():
                pltpu.sync_copy(src_hbm.at[pl.ds(cc, C)], src_v.at[pl.ds(C, C)])
                pltpu.sync_copy(dst_hbm.at[pl.ds(cc, C)], dst_v.at[pl.ds(C, C)])
                pltpu.async_copy(x_hbm.at[src_v.at[pl.ds(C, C)]],
                                 rows_v.at[pl.ds(C, C)], sem_b)

        def wait_rows(par):
            @pl.when(par == 0)
            def _():
                pltpu.make_async_copy(x_hbm.at[src_v.at[pl.ds(0, C)]],
                                      rows_v.at[pl.ds(0, C)], sem_a).wait()

            @pl.when(par == 1)
            def _():
                pltpu.make_async_copy(x_hbm.at[src_v.at[pl.ds(C, C)]],
                                      rows_v.at[pl.ds(C, C)], sem_b).wait()

        open_x(n0)
        zero_ms()

        @pl.when(c_lo < c_hi)
        def _():
            start_chunk(c_lo, 0)

        def chunk_body(c, carry):
            par = lax.rem(c - c_lo, 2)
            base = par * C
            cc = pl.multiple_of(c * C, 8)

            @pl.when(c + 1 < c_hi)
            def _():
                start_chunk(c + 1, 1 - par)

            wait_rows(par)
            j_lo = jnp.maximum(e0 - c * C, 0)
            j_hi = jnp.minimum(e1 - c * C, C)

            def edge_body(j, ecarry):
                (d_cur, wx0, s_v, s_t,
                 a0, a1, a2, a3, a4, a5, a6, a7, av_acc, at_acc) = ecarry
                jb = base + j
                d = _extract(dst_v, jb)
                is_new = d != d_cur

                def do_new(w0):
                    @pl.when((d_cur >= n0) & (d_cur < n1))
                    def _():
                        flush(d_cur, w0, s_v, s_t,
                              (a0, a1, a2, a3, a4, a5, a6, a7))

                    def adv(_, wv):
                        close_win(wv)
                        wv = wv + WX
                        open_x(wv)
                        zero_ms()
                        return wv
                    n_adv = jnp.maximum(d - w0, 0) // WX
                    return lax.fori_loop(0, n_adv, adv, w0)

                wx0 = lax.cond(is_new, do_new, lambda w0: w0, wx0)

                sel = jnp.full((16,), is_new)
                s_v = jnp.where(sel, zero16, s_v)
                s_t = jnp.where(sel, zero16, s_t)
                a0 = jnp.where(sel, zero16, a0)
                a1 = jnp.where(sel, zero16, a1)
                a2 = jnp.where(sel, zero16, a2)
                a3 = jnp.where(sel, zero16, a3)
                a4 = jnp.where(sel, zero16, a4)
                a5 = jnp.where(sel, zero16, a5)
                a6 = jnp.where(sel, zero16, a6)
                a7 = jnp.where(sel, zero16, a7)
                d_cur = d

                r = jnp.maximum(d - wx0, 0)
                r0 = rows_v[jb, pl.ds(0, 16)]
                r1 = rows_v[jb, pl.ds(16, 16)]
                r2 = rows_v[jb, pl.ds(32, 16)]
                r3 = rows_v[jb, pl.ds(48, 16)]
                r4 = rows_v[jb, pl.ds(64, 16)]
                r5 = rows_v[jb, pl.ds(80, 16)]
                r6 = rows_v[jb, pl.ds(96, 16)]
                r7 = rows_v[jb, pl.ds(112, 16)]
                dv = (r0 * xw_v[r, pl.ds(0, 16)] + r1 * xw_v[r, pl.ds(16, 16)]
                      + r2 * xw_v[r, pl.ds(32, 16)]
                      + r3 * xw_v[r, pl.ds(48, 16)])
                dt = (r4 * xw_v[r, pl.ds(64, 16)] + r5 * xw_v[r, pl.ds(80, 16)]
                      + r6 * xw_v[r, pl.ds(96, 16)]
                      + r7 * xw_v[r, pl.ds(112, 16)])
                alpha_v = jnp.sum(dv)
                alpha_t = jnp.sum(dt)

                lane = j - (j // 16) * 16
                lane_sel = iota == lane
                av_acc = jnp.where(lane_sel, _bcast(alpha_v), av_acc)
                at_acc = jnp.where(lane_sel, _bcast(alpha_t), at_acc)

                @pl.when(lane == 15)
                def _():
                    base = pl.multiple_of(j - 15, 16)
                    av_v[pl.ds(base, 16)] = av_acc
                    at_v[pl.ds(base, 16)] = at_acc

                valid = (j >= j_lo) & (j < j_hi)
                av = jnp.where(valid, alpha_v, NEG)
                at = jnp.where(valid, alpha_t, NEG)
                p_v = jnp.exp(_bcast(av))
                p_t = jnp.exp(_bcast(at))
                s_v = s_v + p_v
                s_t = s_t + p_t
                a0 = a0 + p_v * r0
                a1 = a1 + p_v * r1
                a2 = a2 + p_v * r2
                a3 = a3 + p_v * r3
                a4 = a4 + p_t * r4
                a5 = a5 + p_t * r5
                a6 = a6 + p_t * r6
                a7 = a7 + p_t * r7
                return (d_cur, wx0, s_v, s_t,
                        a0, a1, a2, a3, a4, a5, a6, a7, av_acc, at_acc)

            carry = lax.fori_loop(j_lo, C, edge_body, carry)

            @pl.when((c > c_lo) | aligned0)
            def _():
                pltpu.sync_copy(av_v, av_hbm.at[pl.ds(cc, C)])
                pltpu.sync_copy(at_v, at_hbm.at[pl.ds(cc, C)])
            return carry

        init = (jnp.int32(-1), n0,
                zero16, zero16,
                zero16, zero16, zero16, zero16,
                zero16, zero16, zero16, zero16,
                zero16, zero16)
        carry = lax.fori_loop(c_lo, c_hi, chunk_body, init)
        (d_cur, wx0, s_v, s_t,
         a0, a1, a2, a3, a4, a5, a6, a7, _av, _at) = carry

        def do_flush(tok):
            flush(d_cur, wx0, s_v, s_t,
                  (a0, a1, a2, a3, a4, a5, a6, a7))
            return tok
        lax.cond((d_cur >= n0) & (d_cur < n1), do_flush, lambda t: t, 0)

        def drain(_, w0):
            close_win(w0)
            w0 = w0 + WX
            @pl.when(w0 < n1)
            def _():
                open_x(w0)
                zero_ms()
            return w0
        lax.fori_loop(0, (n1 - wx0) // WX, drain, wx0)

    return k(x_cat, srcs, dsts, offs)


def _edge_weights(av, at, ms, conf16, srcs, dsts):
    total_chunks = L2 // C
    kpt = (total_chunks + NW - 1) // NW

    @functools.partial(
        pl.kernel,
        out_type=jax.ShapeDtypeStruct((L2,), _f32),
        mesh=plsc.VectorSubcoreMesh(**_MESH),
        compiler_params=_CP,
        scratch_types=[
            pltpu.VMEM((C,), _i32),
            pltpu.VMEM((C,), _i32),
            pltpu.VMEM((C,), _f32),
            pltpu.VMEM((C,), _f32),
            pltpu.VMEM((C, 16), _f32),
            pltpu.VMEM((C, 16), _f32),
            pltpu.VMEM((C,), _f32),
            pltpu.SemaphoreType.DMA,
            pltpu.SemaphoreType.DMA,
        ],
    )
    def k(av_hbm, at_hbm, ms_hbm, cf_hbm, src_hbm, dst_hbm, w_hbm,
          src_v, dst_v, av_v, at_v, ms_v, cf_v, w_v, sem1, sem2):
        w = _wid()
        c_beg = w * kpt
        c_end = jnp.minimum(c_beg + kpt, total_chunks)
        iota = lax.iota(_i32, 16)

        def chunk_body(c, tok):
            cc = pl.multiple_of(c * C, 8)
            pltpu.sync_copy(src_hbm.at[pl.ds(cc, C)], src_v)
            pltpu.sync_copy(dst_hbm.at[pl.ds(cc, C)], dst_v)
            pltpu.sync_copy(av_hbm.at[pl.ds(cc, C)], av_v)
            pltpu.sync_copy(at_hbm.at[pl.ds(cc, C)], at_v)
            cp1 = pltpu.async_copy(ms_hbm.at[dst_v], ms_v, sem1)
            cp2 = pltpu.async_copy(cf_hbm.at[src_v], cf_v, sem2)
            cp1.wait()
            cp2.wait()
            for g in range(C // 16):
                rows = iota + g * 16
                sv = plsc.load_gather(ms_v, [rows, jnp.zeros((16,), _i32)])
                st = plsc.load_gather(ms_v, [rows, jnp.full((16,), 1, _i32)])
                c0 = plsc.load_gather(cf_v, [rows, jnp.zeros((16,), _i32)])
                c1 = plsc.load_gather(cf_v, [rows, jnp.full((16,), 1, _i32)])
                avv = av_v[pl.ds(g * 16, 16)]
                att = at_v[pl.ds(g * 16, 16)]
                a_v = jnp.exp(avv) / (sv + 1e-16)
                a_t = jnp.exp(att) / (st + 1e-16)
                wv = jnp.maximum(a_v * c0, a_t * c1)
                w_v[pl.ds(g * 16, 16)] = jnp.maximum(wv, 0.0)
            pltpu.sync_copy(w_v, w_hbm.at[pl.ds(cc, C)])
            return tok

        lax.fori_loop(c_beg, c_end, chunk_body, 0)

    return k(av, at, ms, conf16, srcs, dsts)


def _sage(x, srcs, dsts, wts, offs):
    @functools.partial(
        pl.kernel,
        out_type=jax.ShapeDtypeStruct((NPAD, 64), _f32),
        mesh=plsc.VectorSubcoreMesh(**_MESH),
        compiler_params=_CP,
        scratch_types=[
            pltpu.VMEM((64,), _i32),
            pltpu.VMEM((2 * C,), _i32),
            pltpu.VMEM((2 * C + 16,), _i32),
            pltpu.VMEM((2 * C + 16,), _f32),
            pltpu.VMEM((2 * C, 64), _f32),
            pltpu.VMEM((WX, 64), _f32),
            pltpu.SemaphoreType.DMA,
            pltpu.SemaphoreType.DMA,
        ],
    )
    def k(x_hbm, src_hbm, dst_hbm, wt_hbm, off_hbm, out_hbm,
          off_v, src_v, dst_v, wt_v, rows_v, ow_v, sem_a, sem_b):
        w = _wid()
        n0 = w * NPT_F
        n1 = n0 + NPT_F
        pltpu.sync_copy(off_hbm.at[pl.ds(0, 64)], off_v)
        e0 = _extract(off_v, w)
        e1 = _extract(off_v, w + 1)
        c_lo = e0 // C
        c_hi = (e1 + C - 1) // C

        zero16 = jnp.zeros((16,), _f32)

        def zero_out():
            def zb(r, t):
                for kk in range(4):
                    ow_v[r, pl.ds(kk * 16, 16)] = zero16
                return t
            lax.fori_loop(0, WX, zb, 0)

        def close_out(w0):
            @pl.when(w0 < n1)
            def _():
                wa = pl.multiple_of(w0, 8)
                pltpu.sync_copy(ow_v, out_hbm.at[pl.ds(wa, WX), pl.ds(0, 64)])
            zero_out()

        def flush(d_cur, wx0, a):
            r = d_cur - wx0
            for kk in range(4):
                ow_v[r, pl.ds(kk * 16, 16)] = _leaky_v(a[kk])

        def start_chunk(c, par):
            cc = pl.multiple_of(c * C, 8)

            @pl.when(par == 0)
            def _():
                pltpu.sync_copy(src_hbm.at[pl.ds(cc, C)], src_v.at[pl.ds(0, C)])
                pltpu.sync_copy(dst_hbm.at[pl.ds(cc, C)], dst_v.at[pl.ds(0, C)])
                pltpu.sync_copy(wt_hbm.at[pl.ds(cc, C)], wt_v.at[pl.ds(0, C)])
                pltpu.async_copy(x_hbm.at[src_v.at[pl.ds(0, C)]],
                                 rows_v.at[pl.ds(0, C)], sem_a)

            @pl.when(par == 1)
            def _():
                pltpu.sync_copy(src_hbm.at[pl.ds(cc, C)], src_v.at[pl.ds(C, C)])
                pltpu.sync_copy(dst_hbm.at[pl.ds(cc, C)], dst_v.at[pl.ds(C, C)])
                pltpu.sync_copy(wt_hbm.at[pl.ds(cc, C)], wt_v.at[pl.ds(C, C)])
                pltpu.async_copy(x_hbm.at[src_v.at[pl.ds(C, C)]],
                                 rows_v.at[pl.ds(C, C)], sem_b)

        def wait_rows(par):
            @pl.when(par == 0)
            def _():
                pltpu.make_async_copy(x_hbm.at[src_v.at[pl.ds(0, C)]],
                                      rows_v.at[pl.ds(0, C)], sem_a).wait()

            @pl.when(par == 1)
            def _():
                pltpu.make_async_copy(x_hbm.at[src_v.at[pl.ds(C, C)]],
                                      rows_v.at[pl.ds(C, C)], sem_b).wait()

        zero_out()

        @pl.when(c_lo < c_hi)
        def _():
            start_chunk(c_lo, 0)

        def chunk_body(c, carry):
            par = lax.rem(c - c_lo, 2)
            base = par * C

            @pl.when(c + 1 < c_hi)
            def _():
                start_chunk(c + 1, 1 - par)

            wait_rows(par)
            j_lo = jnp.maximum(e0 - c * C, 0)
            j_hi = jnp.minimum(e1 - c * C, C)

            def edge_body(j, ecarry):
                d_cur, wx0, a0, a1, a2, a3 = ecarry
                jb = base + j
                d = _extract(dst_v, jb)
                is_new = d != d_cur

                def do_new(w0):
                    @pl.when(d_cur >= n0)
                    def _():
                        flush(d_cur, w0, (a0, a1, a2, a3))

                    def adv(_, wv):
                        close_out(wv)
                        return wv + WX
                    n_adv = jnp.maximum(d - w0, 0) // WX
                    return lax.fori_loop(0, n_adv, adv, w0)

                wx0 = lax.cond(is_new, do_new, lambda w0: w0, wx0)

                sel = jnp.full((16,), is_new)
                a0 = jnp.where(sel, zero16, a0)
                a1 = jnp.where(sel, zero16, a1)
                a2 = jnp.where(sel, zero16, a2)
                a3 = jnp.where(sel, zero16, a3)
                d_cur = d

                we = _bcast(wt_v[pl.ds(jb, 16)][0])
                a0 = a0 + we * rows_v[jb, pl.ds(0, 16)]
                a1 = a1 + we * rows_v[jb, pl.ds(16, 16)]
                a2 = a2 + we * rows_v[jb, pl.ds(32, 16)]
                a3 = a3 + we * rows_v[jb, pl.ds(48, 16)]
                return (d_cur, wx0, a0, a1, a2, a3)

            return lax.fori_loop(j_lo, j_hi, edge_body, carry)

        init = (jnp.int32(-1), n0, zero16, zero16, zero16, zero16)
        carry = lax.fori_loop(c_lo, c_hi, chunk_body, init)
        d_cur, wx0, a0, a1, a2, a3 = carry

        def do_flush(tok):
            flush(d_cur, wx0, (a0, a1, a2, a3))
            return tok
        lax.cond(d_cur >= n0, do_flush, lambda t: t, 0)

        def drain(_, w0):
            close_out(w0)
            return w0 + WX
        lax.fori_loop(0, (n1 - wx0) // WX, drain, wx0)

    return k(x, srcs, dsts, wts, offs)


def _feats_tc(v_feat, t_feat, v_W, v_b, t_W, t_b):
    vf = jnp.pad(v_feat, ((0, 40960 - NUM_ITEM), (0, 0)))
    tf = jnp.pad(t_feat, ((0, 40960 - NUM_ITEM), (0, 0)))
    vb = jnp.broadcast_to(v_b[None, :], (8, 64))
    tb = jnp.broadcast_to(t_b[None, :], (8, 64))

    def body(vf_ref, tf_ref, vw_ref, tw_ref, vb_ref, tb_ref, o_ref):
        def half(f, wgt, b):
            y = jax.lax.dot_general(f, wgt, (((1,), (1,)), ((), ())),
                                    preferred_element_type=_f32)
            y = y + b[0:1, :]
            y = jnp.where(y >= 0, y, 0.01 * y)
            nrm = jnp.sqrt(jnp.sum(y * y, axis=1, keepdims=True))
            return y / jnp.maximum(nrm, 1e-12)
        o_ref[:, 0:64] = half(vf_ref[...], vw_ref[...], vb_ref[...])
        o_ref[:, 64:128] = half(tf_ref[...], tw_ref[...], tb_ref[...])

    return pl.pallas_call(
        body,
        grid=(40960 // 256,),
        in_specs=[
            pl.BlockSpec((256, 128), lambda i: (i, 0)),
            pl.BlockSpec((256, 128), lambda i: (i, 0)),
            pl.BlockSpec((64, 128), lambda i: (0, 0)),
            pl.BlockSpec((64, 128), lambda i: (0, 0)),
            pl.BlockSpec((8, 64), lambda i: (0, 0)),
            pl.BlockSpec((8, 64), lambda i: (0, 0)),
        ],
        out_specs=pl.BlockSpec((256, 128), lambda i: (i, 0)),
        out_shape=jax.ShapeDtypeStruct((40960, 128), _f32),
    )(vf, tf, v_W, t_W, vb, tb)


def _pref_update_tc(pref_cat, xhat):
    def body(p_ref, x_ref, o_ref):
        y = p_ref[...] + x_ref[...]
        for lo in (0, 64):
            h = y[:, lo:lo + 64]
            nrm = jnp.sqrt(jnp.sum(h * h, axis=1, keepdims=True))
            o_ref[:, lo:lo + 64] = h / jnp.maximum(nrm, 1e-12)

    return pl.pallas_call(
        body,
        grid=(UPAD // 256,),
        in_specs=[
            pl.BlockSpec((256, 128), lambda i: (i, 0)),
            pl.BlockSpec((256, 128), lambda i: (i, 0)),
        ],
        out_specs=pl.BlockSpec((256, 128), lambda i: (i, 0)),
        out_shape=jax.ShapeDtypeStruct((UPAD, 128), _f32),
    )(pref_cat, xhat)


def _l2norm_tc(x):
    def body(x_ref, o_ref):
        y = x_ref[...]
        nrm = jnp.sqrt(jnp.sum(y * y, axis=1, keepdims=True))
        o_ref[...] = y / jnp.maximum(nrm, 1e-12)

    return pl.pallas_call(
        body,
        grid=(NPAD // 256,),
        in_specs=[pl.BlockSpec((256, 64), lambda i: (i, 0))],
        out_specs=pl.BlockSpec((256, 64), lambda i: (i, 0)),
        out_shape=jax.ShapeDtypeStruct((NPAD, 64), _f32),
    )(x)


def _fuse_tc(x0, x1, x2, rep):
    def body(a_ref, b_ref, c_ref, r_ref, o_ref):
        o_ref[:, 0:64] = a_ref[...] + b_ref[...] + c_ref[...]
        o_ref[:, 64:192] = r_ref[...]

    return pl.pallas_call(
        body,
        grid=(NPAD // 256,),
        in_specs=[
            pl.BlockSpec((256, 64), lambda i: (i, 0)),
            pl.BlockSpec((256, 64), lambda i: (i, 0)),
            pl.BlockSpec((256, 64), lambda i: (i, 0)),
            pl.BlockSpec((256, 128), lambda i: (i, 0)),
        ],
        out_specs=pl.BlockSpec((256, 192), lambda i: (i, 0)),
        out_shape=jax.ShapeDtypeStruct((NPAD, 192), _f32),
    )(x0, x1, x2, rep)


def kernel(edge_index, v_feat, t_feat, v_pref, t_pref, v_W, v_b, t_W, t_b,
           id_emb, conf):
    src = edge_index[0].astype(_i32)
    dst = edge_index[1].astype(_i32)

    perm1 = jnp.argsort(dst)
    srcs1 = jnp.pad(src[perm1], (0, L1 - E)).astype(_i32)
    dsts1_u = dst[perm1]
    dsts1 = jnp.pad(dsts1_u, (0, L1 - E)).astype(_i32)
    b_r = jnp.minimum(jnp.arange(33) * NPT_R, NUM_USER)
    off_r = jnp.pad(jnp.searchsorted(dsts1_u, b_r).astype(_i32), (0, 31))

    src2 = jnp.concatenate([src, dst])
    dst2 = jnp.concatenate([dst, src])
    perm2 = jnp.argsort(dst2)
    srcs2 = jnp.pad(src2[perm2], (0, L2 - 2 * E)).astype(_i32)
    dsts2_u = dst2[perm2]
    dsts2 = jnp.pad(dsts2_u, (0, L2 - 2 * E)).astype(_i32)
    b_f = jnp.minimum(jnp.arange(33) * NPT_F, N)
    off_f = jnp.pad(jnp.searchsorted(dsts2_u, b_f).astype(_i32), (0, 31))

    feats_cat = _feats_tc(v_feat, t_feat, v_W, v_b, t_W, t_b)
    pref0 = jnp.pad(jnp.concatenate([v_pref, t_pref], axis=1),
                    ((0, UPAD - NUM_USER), (0, 0)))
    pref_cat = _pref_update_tc(pref0, jnp.zeros((UPAD, 128), _f32))

    pad_tail = jnp.zeros((NPAD - N, 128), _f32)

    for _ in range(NUM_ROUTING):
        x_cat = jnp.concatenate(
            [pref_cat[:NUM_USER], feats_cat[:NUM_ITEM], pad_tail])
        xh = _gat_routing(x_cat, srcs1, dsts1, off_r)
        pref_cat = _pref_update_tc(pref_cat, xh)

    x_cat = jnp.concatenate(
        [pref_cat[:NUM_USER], feats_cat[:NUM_ITEM], pad_tail])
    rep, ms, av, at = _gat_final(x_cat, srcs2, dsts2, off_f)

    conf16 = jnp.zeros((NPAD, 16), _f32).at[:N, 0:2].set(conf)
    wts = _edge_weights(av, at, ms, conf16, srcs2, dsts2)

    x0 = _l2norm_tc(jnp.pad(id_emb, ((0, NPAD - N), (0, 0))))
    x1 = _sage(x0, srcs2, dsts2, wts, off_f)
    x2 = _sage(x1, srcs2, dsts2, wts, off_f)

    out = _fuse_tc(x0, x1, x2, rep)
    return out[:N]


# speedup vs baseline: 14.9483x; 1.0634x over previous
import functools

import jax
import jax.numpy as jnp
from jax import lax
from jax.experimental import pallas as pl
from jax.experimental.pallas import tpu as pltpu
from jax.experimental.pallas import tpu_sc as plsc

NUM_USER = 10000
NUM_ITEM = 40000
N = NUM_USER + NUM_ITEM
E = 800000
NUM_ROUTING = 3

NW = 32
C = 256
WX = 64
NPT_F = 1600
NPAD = NPT_F * NW
NPT_R = 320
UPAD = NPT_R * NW
L1 = (E // C + 1) * C
L2 = (2 * E // C + 1) * C
NEG = -1e30

_MESH = dict(core_axis_name="c", subcore_axis_name="s", num_cores=2,
             num_subcores=16)
_CP = pltpu.CompilerParams(needs_layout_passes=False, use_tc_tiling_on_sc=False)

_f32 = jnp.float32
_i32 = jnp.int32


def _wid():
    return lax.axis_index("s") * 2 + lax.axis_index("c")


def _extract(ref, j):
    return ref[pl.ds(j, 16)][0]


def _bcast(x):
    return jnp.full((16,), x, _f32)


def _leaky_v(x):
    return jnp.where(x >= 0.0, x, 0.01 * x)


def _gat_routing(x_cat, srcs, dsts, offs):
    @functools.partial(
        pl.kernel,
        out_type=jax.ShapeDtypeStruct((UPAD, 128), _f32),
        mesh=plsc.VectorSubcoreMesh(**_MESH),
        compiler_params=_CP,
        scratch_types=[
            pltpu.VMEM((64,), _i32),
            pltpu.VMEM((2 * C,), _i32),
            pltpu.VMEM((2 * C + 16,), _i32),
            pltpu.VMEM((2 * C, 128), _f32),
            pltpu.VMEM((WX, 128), _f32),
            pltpu.VMEM((WX, 128), _f32),
            pltpu.SemaphoreType.DMA,
            pltpu.SemaphoreType.DMA,
        ],
    )
    def k(x_hbm, src_hbm, dst_hbm, off_hbm, out_hbm,
          off_v, src_v, dst_v, rows_v, xw_v, ow_v, sem_a, sem_b):
        w = _wid()
        n0 = w * NPT_R
        n1 = n0 + NPT_R
        pltpu.sync_copy(off_hbm.at[pl.ds(0, 64)], off_v)
        e0 = _extract(off_v, w)
        e1 = _extract(off_v, w + 1)
        c_lo = e0 // C
        c_hi = (e1 + C - 1) // C

        zero16 = jnp.zeros((16,), _f32)

        def open_x(w0):
            wa = pl.multiple_of(w0, 8)
            pltpu.sync_copy(x_hbm.at[pl.ds(wa, WX), pl.ds(0, 128)], xw_v)

        def zero_out():
            def zb(r, t):
                for kk in range(8):
                    ow_v[r, pl.ds(kk * 16, 16)] = zero16
                return t
            lax.fori_loop(0, WX, zb, 0)

        def close_out(w0):
            @pl.when(w0 < n1)
            def _():
                wa = pl.multiple_of(w0, 8)
                pltpu.sync_copy(ow_v, out_hbm.at[pl.ds(wa, WX), pl.ds(0, 128)])
            zero_out()

        def flush(d_cur, wx0, s_v, s_t, a):
            r = d_cur - wx0
            inv_v = _bcast(1.0) / (s_v + 1e-16)
            inv_t = _bcast(1.0) / (s_t + 1e-16)
            for kk in range(4):
                ow_v[r, pl.ds(kk * 16, 16)] = a[kk] * inv_v
            for kk in range(4, 8):
                ow_v[r, pl.ds(kk * 16, 16)] = a[kk] * inv_t

        def start_chunk(c, par):
            cc = pl.multiple_of(c * C, 8)

            @pl.when(par == 0)
            def _():
                pltpu.sync_copy(src_hbm.at[pl.ds(cc, C)], src_v.at[pl.ds(0, C)])
                pltpu.sync_copy(dst_hbm.at[pl.ds(cc, C)], dst_v.at[pl.ds(0, C)])
                pltpu.async_copy(x_hbm.at[src_v.at[pl.ds(0, C)]],
                                 rows_v.at[pl.ds(0, C)], sem_a)

            @pl.when(par == 1)
            def _():
                pltpu.sync_copy(src_hbm.at[pl.ds(cc, C)], src_v.at[pl.ds(C, C)])
                pltpu.sync_copy(dst_hbm.at[pl.ds(cc, C)], dst_v.at[pl.ds(C, C)])
                pltpu.async_copy(x_hbm.at[src_v.at[pl.ds(C, C)]],
                                 rows_v.at[pl.ds(C, C)], sem_b)

        def wait_rows(par):
            @pl.when(par == 0)
            def _():
                pltpu.make_async_copy(x_hbm.at[src_v.at[pl.ds(0, C)]],
                                      rows_v.at[pl.ds(0, C)], sem_a).wait()

            @pl.when(par == 1)
            def _():
                pltpu.make_async_copy(x_hbm.at[src_v.at[pl.ds(C, C)]],
                                      rows_v.at[pl.ds(C, C)], sem_b).wait()

        open_x(n0)
        zero_out()

        @pl.when(c_lo < c_hi)
        def _():
            start_chunk(c_lo, 0)

        def chunk_body(c, carry):
            par = lax.rem(c - c_lo, 2)
            base = par * C

            @pl.when(c + 1 < c_hi)
            def _():
                start_chunk(c + 1, 1 - par)

            wait_rows(par)
            j_lo = jnp.maximum(e0 - c * C, 0)
            j_hi = jnp.minimum(e1 - c * C, C)

            def edge_body(j, ecarry):
                (d_cur, wx0, s_v, s_t,
                 a0, a1, a2, a3, a4, a5, a6, a7) = ecarry
                jb = base + j
                d = _extract(dst_v, jb)
                is_new = d != d_cur

                def do_new(w0):
                    @pl.when(d_cur >= n0)
                    def _():
                        flush(d_cur, w0, s_v, s_t,
                              (a0, a1, a2, a3, a4, a5, a6, a7))

                    def adv(_, wv):
                        close_out(wv)
                        wv = wv + WX
                        open_x(wv)
                        return wv
                    n_adv = jnp.maximum(d - w0, 0) // WX
                    return lax.fori_loop(0, n_adv, adv, w0)

                wx0 = lax.cond(is_new, do_new, lambda w0: w0, wx0)

                sel = jnp.full((16,), is_new)
                s_v = jnp.where(sel, zero16, s_v)
                s_t = jnp.where(sel, zero16, s_t)
                a0 = jnp.where(sel, zero16, a0)
                a1 = jnp.where(sel, zero16, a1)
                a2 = jnp.where(sel, zero16, a2)
                a3 = jnp.where(sel, zero16, a3)
                a4 = jnp.where(sel, zero16, a4)
                a5 = jnp.where(sel, zero16, a5)
                a6 = jnp.where(sel, zero16, a6)
                a7 = jnp.where(sel, zero16, a7)
                d_cur = d

                r = jnp.maximum(d - wx0, 0)
                r0 = rows_v[jb, pl.ds(0, 16)]
                r1 = rows_v[jb, pl.ds(16, 16)]
                r2 = rows_v[jb, pl.ds(32, 16)]
                r3 = rows_v[jb, pl.ds(48, 16)]
                r4 = rows_v[jb, pl.ds(64, 16)]
                r5 = rows_v[jb, pl.ds(80, 16)]
                r6 = rows_v[jb, pl.ds(96, 16)]
                r7 = rows_v[jb, pl.ds(112, 16)]
                dv = (r0 * xw_v[r, pl.ds(0, 16)] + r1 * xw_v[r, pl.ds(16, 16)]
                      + r2 * xw_v[r, pl.ds(32, 16)]
                      + r3 * xw_v[r, pl.ds(48, 16)])
                dt = (r4 * xw_v[r, pl.ds(64, 16)] + r5 * xw_v[r, pl.ds(80, 16)]
                      + r6 * xw_v[r, pl.ds(96, 16)]
                      + r7 * xw_v[r, pl.ds(112, 16)])
                valid = (j >= j_lo) & (j < j_hi)
                av = jnp.where(valid, jnp.sum(dv), NEG)
                at = jnp.where(valid, jnp.sum(dt), NEG)
                p_v = jnp.exp(_bcast(av))
                p_t = jnp.exp(_bcast(at))
                s_v = s_v + p_v
                s_t = s_t + p_t
                a0 = a0 + p_v * r0
                a1 = a1 + p_v * r1
                a2 = a2 + p_v * r2
                a3 = a3 + p_v * r3
                a4 = a4 + p_t * r4
                a5 = a5 + p_t * r5
                a6 = a6 + p_t * r6
                a7 = a7 + p_t * r7
                return (d_cur, wx0, s_v, s_t,
                        a0, a1, a2, a3, a4, a5, a6, a7)

            return lax.fori_loop(j_lo, j_hi, edge_body, carry)

        init = (jnp.int32(-1), n0,
                zero16, zero16,
                zero16, zero16, zero16, zero16,
                zero16, zero16, zero16, zero16)
        carry = lax.fori_loop(c_lo, c_hi, chunk_body, init)
        (d_cur, wx0, s_v, s_t,
         a0, a1, a2, a3, a4, a5, a6, a7) = carry

        def do_flush(tok):
            flush(d_cur, wx0, s_v, s_t, (a0, a1, a2, a3, a4, a5, a6, a7))
            return tok
        lax.cond(d_cur >= n0, do_flush, lambda t: t, 0)

        def drain(_, w0):
            close_out(w0)
            return w0 + WX
        lax.fori_loop(0, (n1 - wx0) // WX, drain, wx0)

    return k(x_cat, srcs, dsts, offs)


def _gat_final(x_cat, srcs, dsts, offs):
    @functools.partial(
        pl.kernel,
        out_type=(
            jax.ShapeDtypeStruct((NPAD, 128), _f32),
            jax.ShapeDtypeStruct((NPAD, 16), _f32),
            jax.ShapeDtypeStruct((L2,), _f32),
            jax.ShapeDtypeStruct((L2,), _f32),
        ),
        mesh=plsc.VectorSubcoreMesh(**_MESH),
        compiler_params=_CP,
        scratch_types=[
            pltpu.VMEM((64,), _i32),
            pltpu.VMEM((2 * C,), _i32),
            pltpu.VMEM((2 * C + 16,), _i32),
            pltpu.VMEM((2 * C, 128), _f32),
            pltpu.VMEM((WX, 128), _f32),
            pltpu.VMEM((WX, 16), _f32),
            pltpu.VMEM((C,), _f32),
            pltpu.VMEM((C,), _f32),
            pltpu.SemaphoreType.DMA,
            pltpu.SemaphoreType.DMA,
        ],
    )
    def k(x_hbm, src_hbm, dst_hbm, off_hbm,
          rep_hbm, ms_hbm, av_hbm, at_hbm,
          off_v, src_v, dst_v, rows_v, xw_v, msw_v, av_v, at_v, sem_a, sem_b):
        w = _wid()
        n0 = w * NPT_F
        n1 = n0 + NPT_F
        pltpu.sync_copy(off_hbm.at[pl.ds(0, 64)], off_v)
        e0 = _extract(off_v, w)
        e1 = _extract(off_v, w + 1)
        c_lo = e0 // C
        c_hi = (e1 + C - 1) // C
        aligned0 = e0 == c_lo * C

        zero16 = jnp.zeros((16,), _f32)
        iota = lax.iota(_i32, 16)

        def open_x(w0):
            wa = pl.multiple_of(w0, 8)
            pltpu.sync_copy(x_hbm.at[pl.ds(wa, WX), pl.ds(0, 128)], xw_v)

        def zero_ms():
            def zb(r, t):
                msw_v[r, pl.ds(0, 16)] = zero16
                return t
            lax.fori_loop(0, WX, zb, 0)

        def close_win(w0):
            @pl.when(w0 < n1)
            def _():
                wa = pl.multiple_of(w0, 8)
                pltpu.sync_copy(xw_v, rep_hbm.at[pl.ds(wa, WX), pl.ds(0, 128)])
                pltpu.sync_copy(msw_v, ms_hbm.at[pl.ds(wa, WX), pl.ds(0, 16)])

        def flush(d_cur, wx0, s_v, s_t, a):
            r = d_cur - wx0
            inv_v = _bcast(1.0) / (s_v + 1e-16)
            inv_t = _bcast(1.0) / (s_t + 1e-16)
            for kk in range(4):
                x_row = xw_v[r, pl.ds(kk * 16, 16)]
                xw_v[r, pl.ds(kk * 16, 16)] = x_row + _leaky_v(a[kk] * inv_v)
            for kk in range(4, 8):
                x_row = xw_v[r, pl.ds(kk * 16, 16)]
                xw_v[r, pl.ds(kk * 16, 16)] = x_row + _leaky_v(a[kk] * inv_t)
            msrow = jnp.where(iota == 0, s_v, jnp.where(iota == 1, s_t, zero16))
            msw_v[r, pl.ds(0, 16)] = msrow

        def start_chunk(c, par):
            cc = pl.multiple_of(c * C, 8)

            @pl.when(par == 0)
            def _():
                pltpu.sync_copy(src_hbm.at[pl.ds(cc, C)], src_v.at[pl.ds(0, C)])
                pltpu.sync_copy(dst_hbm.at[pl.ds(cc, C)], dst_v.at[pl.ds(0, C)])
                pltpu.async_copy(x_hbm.at[src_v.at[pl.ds(0, C)]],
                                 rows_v.at[pl.ds(0, C)], sem_a)

            @pl.when(par == 1)
            def _():
                pltpu.sync_copy(src_hbm.at[pl.ds(cc, C)], src_v.at[pl.ds(C, C)])
                pltpu.sync_copy(dst_hbm.at[pl.ds(cc, C)], dst_v.at[pl.ds(C, C)])
                pltpu.async_copy(x_hbm.at[src_v.at[pl.ds(C, C)]],
                                 rows_v.at[pl.ds(C, C)], sem_b)

        def wait_rows(par):
            @pl.when(par == 0)
            def _():
                pltpu.make_async_copy(x_hbm.at[src_v.at[pl.ds(0, C)]],
                                      rows_v.at[pl.ds(0, C)], sem_a).wait()

            @pl.when(par == 1)
            def _():
                pltpu.make_async_copy(x_hbm.at[src_v.at[pl.ds(C, C)]],
                                      rows_v.at[pl.ds(C, C)], sem_b).wait()

        open_x(n0)
        zero_ms()

        @pl.when(c_lo < c_hi)
        def _():
            start_chunk(c_lo, 0)

        def chunk_body(c, carry):
            par = lax.rem(c - c_lo, 2)
            base = par * C
            cc = pl.multiple_of(c * C, 8)

            @pl.when(c + 1 < c_hi)
            def _():
                start_chunk(c + 1, 1 - par)

            wait_rows(par)
            j_lo = jnp.maximum(e0 - c * C, 0)
            j_hi = jnp.minimum(e1 - c * C, C)

            def edge_body(j, ecarry):
                (d_cur, wx0, s_v, s_t,
                 a0, a1, a2, a3, a4, a5, a6, a7, av_acc, at_acc) = ecarry
                jb = base + j
                d = _extract(dst_v, jb)
                is_new = d != d_cur

                def do_new(w0):
                    @pl.when((d_cur >= n0) & (d_cur < n1))
                    def _():
                        flush(d_cur, w0, s_v, s_t,
                              (a0, a1, a2, a3, a4, a5, a6, a7))

                    def adv(_, wv):
                        close_win(wv)
                        wv = wv + WX
                        open_x(wv)
                        zero_ms()
                        return wv
                    n_adv = jnp.maximum(d - w0, 0) // WX
                    return lax.fori_loop(0, n_adv, adv, w0)

                wx0 = lax.cond(is_new, do_new, lambda w0: w0, wx0)

                sel = jnp.full((16,), is_new)
                s_v = jnp.where(sel, zero16, s_v)
                s_t = jnp.where(sel, zero16, s_t)
                a0 = jnp.where(sel, zero16, a0)
                a1 = jnp.where(sel, zero16, a1)
                a2 = jnp.where(sel, zero16, a2)
                a3 = jnp.where(sel, zero16, a3)
                a4 = jnp.where(sel, zero16, a4)
                a5 = jnp.where(sel, zero16, a5)
                a6 = jnp.where(sel, zero16, a6)
                a7 = jnp.where(sel, zero16, a7)
                d_cur = d

                r = jnp.maximum(d - wx0, 0)
                r0 = rows_v[jb, pl.ds(0, 16)]
                r1 = rows_v[jb, pl.ds(16, 16)]
                r2 = rows_v[jb, pl.ds(32, 16)]
                r3 = rows_v[jb, pl.ds(48, 16)]
                r4 = rows_v[jb, pl.ds(64, 16)]
                r5 = rows_v[jb, pl.ds(80, 16)]
                r6 = rows_v[jb, pl.ds(96, 16)]
                r7 = rows_v[jb, pl.ds(112, 16)]
                dv = (r0 * xw_v[r, pl.ds(0, 16)] + r1 * xw_v[r, pl.ds(16, 16)]
                      + r2 * xw_v[r, pl.ds(32, 16)]
                      + r3 * xw_v[r, pl.ds(48, 16)])
                dt = (r4 * xw_v[r, pl.ds(64, 16)] + r5 * xw_v[r, pl.ds(80, 16)]
                      + r6 * xw_v[r, pl.ds(96, 16)]
                      + r7 * xw_v[r, pl.ds(112, 16)])
                alpha_v = jnp.sum(dv)
                alpha_t = jnp.sum(dt)

                lane = j - (j // 16) * 16
                lane_sel = iota == lane
                av_acc = jnp.where(lane_sel, _bcast(alpha_v), av_acc)
                at_acc = jnp.where(lane_sel, _bcast(alpha_t), at_acc)

                @pl.when(lane == 15)
                def _():
                    base = pl.multiple_of(j - 15, 16)
                    av_v[pl.ds(base, 16)] = av_acc
                    at_v[pl.ds(base, 16)] = at_acc

                valid = (j >= j_lo) & (j < j_hi)
                av = jnp.where(valid, alpha_v, NEG)
                at = jnp.where(valid, alpha_t, NEG)
                p_v = jnp.exp(_bcast(av))
                p_t = jnp.exp(_bcast(at))
                s_v = s_v + p_v
                s_t = s_t + p_t
                a0 = a0 + p_v * r0
                a1 = a1 + p_v * r1
                a2 = a2 + p_v * r2
                a3 = a3 + p_v * r3
                a4 = a4 + p_t * r4
                a5 = a5 + p_t * r5
                a6 = a6 + p_t * r6
                a7 = a7 + p_t * r7
                return (d_cur, wx0, s_v, s_t,
                        a0, a1, a2, a3, a4, a5, a6, a7, av_acc, at_acc)

            carry = lax.fori_loop(j_lo, C, edge_body, carry)

            @pl.when((c > c_lo) | aligned0)
            def _():
                pltpu.sync_copy(av_v, av_hbm.at[pl.ds(cc, C)])
                pltpu.sync_copy(at_v, at_hbm.at[pl.ds(cc, C)])
            return carry

        init = (jnp.int32(-1), n0,
                zero16, zero16,
                zero16, zero16, zero16, zero16,
                zero16, zero16, zero16, zero16,
                zero16, zero16)
        carry = lax.fori_loop(c_lo, c_hi, chunk_body, init)
        (d_cur, wx0, s_v, s_t,
         a0, a1, a2, a3, a4, a5, a6, a7, _av, _at) = carry

        def do_flush(tok):
            flush(d_cur, wx0, s_v, s_t,
                  (a0, a1, a2, a3, a4, a5, a6, a7))
            return tok
        lax.cond((d_cur >= n0) & (d_cur < n1), do_flush, lambda t: t, 0)

        def drain(_, w0):
            close_win(w0)
            w0 = w0 + WX
            @pl.when(w0 < n1)
            def _():
                open_x(w0)
                zero_ms()
            return w0
        lax.fori_loop(0, (n1 - wx0) // WX, drain, wx0)

    return k(x_cat, srcs, dsts, offs)


def _edge_weights(av, at, ms, conf16, srcs, dsts):
    total_chunks = L2 // C
    kpt = (total_chunks + NW - 1) // NW

    @functools.partial(
        pl.kernel,
        out_type=jax.ShapeDtypeStruct((L2,), _f32),
        mesh=plsc.VectorSubcoreMesh(**_MESH),
        compiler_params=_CP,
        scratch_types=[
            pltpu.VMEM((C,), _i32),
            pltpu.VMEM((C,), _i32),
            pltpu.VMEM((C,), _f32),
            pltpu.VMEM((C,), _f32),
            pltpu.VMEM((C, 16), _f32),
            pltpu.VMEM((C, 16), _f32),
            pltpu.VMEM((C,), _f32),
            pltpu.SemaphoreType.DMA,
            pltpu.SemaphoreType.DMA,
        ],
    )
    def k(av_hbm, at_hbm, ms_hbm, cf_hbm, src_hbm, dst_hbm, w_hbm,
          src_v, dst_v, av_v, at_v, ms_v, cf_v, w_v, sem1, sem2):
        w = _wid()
        c_beg = w * kpt
        c_end = jnp.minimum(c_beg + kpt, total_chunks)
        iota = lax.iota(_i32, 16)

        def chunk_body(c, tok):
            cc = pl.multiple_of(c * C, 8)
            pltpu.sync_copy(src_hbm.at[pl.ds(cc, C)], src_v)
            pltpu.sync_copy(dst_hbm.at[pl.ds(cc, C)], dst_v)
            pltpu.sync_copy(av_hbm.at[pl.ds(cc, C)], av_v)
            pltpu.sync_copy(at_hbm.at[pl.ds(cc, C)], at_v)
            cp1 = pltpu.async_copy(ms_hbm.at[dst_v], ms_v, sem1)
            cp2 = pltpu.async_copy(cf_hbm.at[src_v], cf_v, sem2)
            cp1.wait()
            cp2.wait()
            for g in range(C // 16):
                rows = iota + g * 16
                sv = plsc.load_gather(ms_v, [rows, jnp.zeros((16,), _i32)])
                st = plsc.load_gather(ms_v, [rows, jnp.full((16,), 1, _i32)])
                c0 = plsc.load_gather(cf_v, [rows, jnp.zeros((16,), _i32)])
                c1 = plsc.load_gather(cf_v, [rows, jnp.full((16,), 1, _i32)])
                avv = av_v[pl.ds(g * 16, 16)]
                att = at_v[pl.ds(g * 16, 16)]
                a_v = jnp.exp(avv) / (sv + 1e-16)
                a_t = jnp.exp(att) / (st + 1e-16)
                wv = jnp.maximum(a_v * c0, a_t * c1)
                w_v[pl.ds(g * 16, 16)] = jnp.maximum(wv, 0.0)
            pltpu.sync_copy(w_v, w_hbm.at[pl.ds(cc, C)])
            return tok

        lax.fori_loop(c_beg, c_end, chunk_body, 0)

    return k(av, at, ms, conf16, srcs, dsts)


def _sage(x, srcs, dsts, wts, offs):
    @functools.partial(
        pl.kernel,
        out_type=jax.ShapeDtypeStruct((NPAD, 64), _f32),
        mesh=plsc.VectorSubcoreMesh(**_MESH),
        compiler_params=_CP,
        scratch_types=[
            pltpu.VMEM((64,), _i32),
            pltpu.VMEM((2 * C,), _i32),
            pltpu.VMEM((2 * C + 16,), _i32),
            pltpu.VMEM((2 * C + 16,), _f32),
            pltpu.VMEM((2 * C, 64), _f32),
            pltpu.VMEM((WX, 64), _f32),
            pltpu.SemaphoreType.DMA,
            pltpu.SemaphoreType.DMA,
        ],
    )
    def k(x_hbm, src_hbm, dst_hbm, wt_hbm, off_hbm, out_hbm,
          off_v, src_v, dst_v, wt_v, rows_v, ow_v, sem_a, sem_b):
        w = _wid()
        n0 = w * NPT_F
        n1 = n0 + NPT_F
        pltpu.sync_copy(off_hbm.at[pl.ds(0, 64)], off_v)
        e0 = _extract(off_v, w)
        e1 = _extract(off_v, w + 1)
        c_lo = e0 // C
        c_hi = (e1 + C - 1) // C

        zero16 = jnp.zeros((16,), _f32)

        def zero_out():
            def zb(r, t):
                for kk in range(4):
                    ow_v[r, pl.ds(kk * 16, 16)] = zero16
                return t
            lax.fori_loop(0, WX, zb, 0)

        def close_out(w0):
            @pl.when(w0 < n1)
            def _():
                wa = pl.multiple_of(w0, 8)
                pltpu.sync_copy(ow_v, out_hbm.at[pl.ds(wa, WX), pl.ds(0, 64)])
            zero_out()

        def flush(d_cur, wx0, a):
            r = d_cur - wx0
            for kk in range(4):
                ow_v[r, pl.ds(kk * 16, 16)] = _leaky_v(a[kk])

        def start_chunk(c, par):
            cc = pl.multiple_of(c * C, 8)

            @pl.when(par == 0)
            def _():
                pltpu.sync_copy(src_hbm.at[pl.ds(cc, C)], src_v.at[pl.ds(0, C)])
                pltpu.sync_copy(dst_hbm.at[pl.ds(cc, C)], dst_v.at[pl.ds(0, C)])
                pltpu.sync_copy(wt_hbm.at[pl.ds(cc, C)], wt_v.at[pl.ds(0, C)])
                pltpu.async_copy(x_hbm.at[src_v.at[pl.ds(0, C)]],
                                 rows_v.at[pl.ds(0, C)], sem_a)

            @pl.when(par == 1)
            def _():
                pltpu.sync_copy(src_hbm.at[pl.ds(cc, C)], src_v.at[pl.ds(C, C)])
                pltpu.sync_copy(dst_hbm.at[pl.ds(cc, C)], dst_v.at[pl.ds(C, C)])
                pltpu.sync_copy(wt_hbm.at[pl.ds(cc, C)], wt_v.at[pl.ds(C, C)])
                pltpu.async_copy(x_hbm.at[src_v.at[pl.ds(C, C)]],
                                 rows_v.at[pl.ds(C, C)], sem_b)

        def wait_rows(par):
            @pl.when(par == 0)
            def _():
                pltpu.make_async_copy(x_hbm.at[src_v.at[pl.ds(0, C)]],
                                      rows_v.at[pl.ds(0, C)], sem_a).wait()

            @pl.when(par == 1)
            def _():
                pltpu.make_async_copy(x_hbm.at[src_v.at[pl.ds(C, C)]],
                                      rows_v.at[pl.ds(C, C)], sem_b).wait()

        zero_out()

        @pl.when(c_lo < c_hi)
        def _():
            start_chunk(c_lo, 0)

        def chunk_body(c, carry):
            par = lax.rem(c - c_lo, 2)
            base = par * C

            @pl.when(c + 1 < c_hi)
            def _():
                start_chunk(c + 1, 1 - par)

            wait_rows(par)
            j_lo = jnp.maximum(e0 - c * C, 0)
            j_hi = jnp.minimum(e1 - c * C, C)

            def edge_body(j, ecarry):
                d_cur, wx0, a0, a1, a2, a3 = ecarry
                jb = base + j
                d = _extract(dst_v, jb)
                is_new = d != d_cur

                def do_new(w0):
                    @pl.when(d_cur >= n0)
                    def _():
                        flush(d_cur, w0, (a0, a1, a2, a3))

                    def adv(_, wv):
                        close_out(wv)
                        return wv + WX
                    n_adv = jnp.maximum(d - w0, 0) // WX
                    return lax.fori_loop(0, n_adv, adv, w0)

                wx0 = lax.cond(is_new, do_new, lambda w0: w0, wx0)

                sel = jnp.full((16,), is_new)
                a0 = jnp.where(sel, zero16, a0)
                a1 = jnp.where(sel, zero16, a1)
                a2 = jnp.where(sel, zero16, a2)
                a3 = jnp.where(sel, zero16, a3)
                d_cur = d

                we = _bcast(wt_v[pl.ds(jb, 16)][0])
                a0 = a0 + we * rows_v[jb, pl.ds(0, 16)]
                a1 = a1 + we * rows_v[jb, pl.ds(16, 16)]
                a2 = a2 + we * rows_v[jb, pl.ds(32, 16)]
                a3 = a3 + we * rows_v[jb, pl.ds(48, 16)]
                return (d_cur, wx0, a0, a1, a2, a3)

            return lax.fori_loop(j_lo, j_hi, edge_body, carry)

        init = (jnp.int32(-1), n0, zero16, zero16, zero16, zero16)
        carry = lax.fori_loop(c_lo, c_hi, chunk_body, init)
        d_cur, wx0, a0, a1, a2, a3 = carry

        def do_flush(tok):
            flush(d_cur, wx0, (a0, a1, a2, a3))
            return tok
        lax.cond(d_cur >= n0, do_flush, lambda t: t, 0)

        def drain(_, w0):
            close_out(w0)
            return w0 + WX
        lax.fori_loop(0, (n1 - wx0) // WX, drain, wx0)

    return k(x, srcs, dsts, wts, offs)


def _feats_tc(v_feat, t_feat, v_W, v_b, t_W, t_b):
    vf = jnp.pad(v_feat, ((0, 40960 - NUM_ITEM), (0, 0)))
    tf = jnp.pad(t_feat, ((0, 40960 - NUM_ITEM), (0, 0)))
    vb = jnp.broadcast_to(v_b[None, :], (8, 64))
    tb = jnp.broadcast_to(t_b[None, :], (8, 64))

    def body(vf_ref, tf_ref, vw_ref, tw_ref, vb_ref, tb_ref, o_ref):
        def half(f, wgt, b):
            y = jax.lax.dot_general(f, wgt, (((1,), (1,)), ((), ())),
                                    preferred_element_type=_f32)
            y = y + b[0:1, :]
            y = jnp.where(y >= 0, y, 0.01 * y)
            nrm = jnp.sqrt(jnp.sum(y * y, axis=1, keepdims=True))
            return y / jnp.maximum(nrm, 1e-12)
        o_ref[:, 0:64] = half(vf_ref[...], vw_ref[...], vb_ref[...])
        o_ref[:, 64:128] = half(tf_ref[...], tw_ref[...], tb_ref[...])

    return pl.pallas_call(
        body,
        grid=(40960 // 256,),
        in_specs=[
            pl.BlockSpec((256, 128), lambda i: (i, 0)),
            pl.BlockSpec((256, 128), lambda i: (i, 0)),
            pl.BlockSpec((64, 128), lambda i: (0, 0)),
            pl.BlockSpec((64, 128), lambda i: (0, 0)),
            pl.BlockSpec((8, 64), lambda i: (0, 0)),
            pl.BlockSpec((8, 64), lambda i: (0, 0)),
        ],
        out_specs=pl.BlockSpec((256, 128), lambda i: (i, 0)),
        out_shape=jax.ShapeDtypeStruct((40960, 128), _f32),
    )(vf, tf, v_W, t_W, vb, tb)


def _pref_update_tc(pref_cat, xhat):
    def body(p_ref, x_ref, o_ref):
        y = p_ref[...] + x_ref[...]
        for lo in (0, 64):
            h = y[:, lo:lo + 64]
            nrm = jnp.sqrt(jnp.sum(h * h, axis=1, keepdims=True))
            o_ref[:, lo:lo + 64] = h / jnp.maximum(nrm, 1e-12)

    return pl.pallas_call(
        body,
        grid=(UPAD // 256,),
        in_specs=[
            pl.BlockSpec((256, 128), lambda i: (i, 0)),
            pl.BlockSpec((256, 128), lambda i: (i, 0)),
        ],
        out_specs=pl.BlockSpec((256, 128), lambda i: (i, 0)),
        out_shape=jax.ShapeDtypeStruct((UPAD, 128), _f32),
    )(pref_cat, xhat)


def _l2norm_tc(x):
    def body(x_ref, o_ref):
        y = x_ref[...]
        nrm = jnp.sqrt(jnp.sum(y * y, axis=1, keepdims=True))
        o_ref[...] = y / jnp.maximum(nrm, 1e-12)

    return pl.pallas_call(
        body,
        grid=(NPAD // 256,),
        in_specs=[pl.BlockSpec((256, 64), lambda i: (i, 0))],
        out_specs=pl.BlockSpec((256, 64), lambda i: (i, 0)),
        out_shape=jax.ShapeDtypeStruct((NPAD, 64), _f32),
    )(x)


def _fuse_tc(x0, x1, x2, rep):
    def body(a_ref, b_ref, c_ref, r_ref, o_ref):
        o_ref[:, 0:64] = a_ref[...] + b_ref[...] + c_ref[...]
        o_ref[:, 64:192] = r_ref[...]

    return pl.pallas_call(
        body,
        grid=(NPAD // 256,),
        in_specs=[
            pl.BlockSpec((256, 64), lambda i: (i, 0)),
            pl.BlockSpec((256, 64), lambda i: (i, 0)),
            pl.BlockSpec((256, 64), lambda i: (i, 0)),
            pl.BlockSpec((256, 128), lambda i: (i, 0)),
        ],
        out_specs=pl.BlockSpec((256, 192), lambda i: (i, 0)),
        out_shape=jax.ShapeDtypeStruct((NPAD, 192), _f32),
    )(x0, x1, x2, rep)


def kernel(edge_index, v_feat, t_feat, v_pref, t_pref, v_W, v_b, t_W, t_b,
           id_emb, conf):
    src = edge_index[0].astype(_i32)
    dst = edge_index[1].astype(_i32)

    dsts1_u, srcs1_u = lax.sort((dst, src), num_keys=1, is_stable=False)
    srcs1 = jnp.pad(srcs1_u, (0, L1 - E))
    dsts1 = jnp.pad(dsts1_u, (0, L1 - E))
    b_r = jnp.minimum(jnp.arange(33) * NPT_R, NUM_USER)
    off_r = jnp.pad(jnp.searchsorted(dsts1_u, b_r).astype(_i32), (0, 31))

    src2 = jnp.concatenate([src, dst])
    dst2 = jnp.concatenate([dst, src])
    dsts2_u, srcs2_u = lax.sort((dst2, src2), num_keys=1, is_stable=False)
    srcs2 = jnp.pad(srcs2_u, (0, L2 - 2 * E))
    dsts2 = jnp.pad(dsts2_u, (0, L2 - 2 * E))
    b_f = jnp.minimum(jnp.arange(33) * NPT_F, N)
    off_f = jnp.pad(jnp.searchsorted(dsts2_u, b_f).astype(_i32), (0, 31))

    feats_cat = _feats_tc(v_feat, t_feat, v_W, v_b, t_W, t_b)
    pref0 = jnp.pad(jnp.concatenate([v_pref, t_pref], axis=1),
                    ((0, UPAD - NUM_USER), (0, 0)))
    pref_cat = _pref_update_tc(pref0, jnp.zeros((UPAD, 128), _f32))

    pad_tail = jnp.zeros((NPAD - N, 128), _f32)

    for _ in range(NUM_ROUTING):
        x_cat = jnp.concatenate(
            [pref_cat[:NUM_USER], feats_cat[:NUM_ITEM], pad_tail])
        xh = _gat_routing(x_cat, srcs1, dsts1, off_r)
        pref_cat = _pref_update_tc(pref_cat, xh)

    x_cat = jnp.concatenate(
        [pref_cat[:NUM_USER], feats_cat[:NUM_ITEM], pad_tail])
    rep, ms, av, at = _gat_final(x_cat, srcs2, dsts2, off_f)

    conf16 = jnp.zeros((NPAD, 16), _f32).at[:N, 0:2].set(conf)
    wts = _edge_weights(av, at, ms, conf16, srcs2, dsts2)

    x0 = _l2norm_tc(jnp.pad(id_emb, ((0, NPAD - N), (0, 0))))
    x1 = _sage(x0, srcs2, dsts2, wts, off_f)
    x2 = _sage(x1, srcs2, dsts2, wts, off_f)

    out = _fuse_tc(x0, x1, x2, rep)
    return out[:N]
